# Initial kernel scaffold; baseline (speedup 1.0000x reference)
import jax
import jax.numpy as jnp
from jax.experimental import pallas as pl


def kernel(x_item, x_user, edge_index_iu, edge_index_ui, y_emb, emb_weight, W_rel1_iu, b_rel1_iu, W_root1_iu, W_rel1_ui, b_rel1_ui, W_root1_ui, gamma_item, beta_item, gamma_user, beta_user, W_rel2_iu, b_rel2_iu, W_root2_iu, W_rel2_ui, b_rel2_ui, W_root2_ui, lin_W, lin_b):
    raise NotImplementedError("write your pallas kernel here")


# speedup vs baseline: 3.6390x; 3.6390x over previous
import jax
import jax.numpy as jnp
from jax import lax
from jax.experimental import pallas as pl
from jax.experimental.pallas import tpu as pltpu
from jax.experimental.pallas import tpu_sc as plsc

N = 10000
E = 160000
D = 256
H = 512

NC = 2
NT = 16
HALF = N // NC
QR = (2504, 2496)
TBLR = 2560
ZPT = TBLR // NT
EPT = E // NT
CH = 128
CHB = 7
MAXCH = 80
PIECE = 2000


def _flush(table, out_hbm, t, gbase, size):
    @pl.when(t < 12)
    def _():
        pltpu.sync_copy(table.at[pl.ds(t * 208, 208)],
                        out_hbm.at[pl.ds(gbase + t * 208, 208)])
    if size == 2504:
        @pl.when(t == 12)
        def _():
            pltpu.sync_copy(table.at[pl.ds(2496, 8)],
                            out_hbm.at[pl.ds(gbase + 2496, 8)])


def _compact2(t, base, srcv_hbm, dstv_hbm, srcraw, dstraw, bufs):
    (srcA, dstA), (srcB, dstB) = bufs
    baseB = base + QR[0]

    def piece(pi, carry):
        pltpu.sync_copy(srcv_hbm.at[pl.ds(t * EPT + pi * PIECE, PIECE)],
                        srcraw)
        pltpu.sync_copy(dstv_hbm.at[pl.ds(t * EPT + pi * PIECE, PIECE)],
                        dstraw)

        def cbody(j, carry):
            cntA, cntB = carry
            sv = srcraw[pl.ds(j * 16, 16)]
            dv = dstraw[pl.ds(j * 16, 16)]
            dA = dv - base
            mA = (dA >= 0) & (dA < QR[0])
            posA = plsc.cumsum(mA.astype(jnp.int32))
            iA = cntA + posA - 1
            plsc.store_scatter(srcA, [iA >> CHB, iA & (CH - 1)], sv, mask=mA)
            plsc.store_scatter(dstA, [iA >> CHB, iA & (CH - 1)], dA, mask=mA)
            dB = dv - baseB
            mB = (dB >= 0) & (dB < QR[1])
            posB = plsc.cumsum(mB.astype(jnp.int32))
            iB = cntB + posB - 1
            plsc.store_scatter(srcB, [iB >> CHB, iB & (CH - 1)], sv, mask=mB)
            plsc.store_scatter(dstB, [iB >> CHB, iB & (CH - 1)], dB, mask=mB)
            return (cntA + posA[15], cntB + posB[15])

        return lax.fori_loop(0, PIECE // 16, cbody, carry)

    cntA, cntB = lax.fori_loop(0, EPT // PIECE, piece,
                               (jnp.int32(0), jnp.int32(0)))
    lane = lax.iota(jnp.int32, 16)
    for (sbuf, dbuf, cnt, q) in ((srcA, dstA, cntA, 0), (srcB, dstB, cntB, 1)):
        for k in range(CH // 16):
            i2 = cnt + k * 16 + lane
            plsc.store_scatter(sbuf, [i2 >> CHB, i2 & (CH - 1)], lane + k * 16)
            plsc.store_scatter(dbuf, [i2 >> CHB, i2 & (CH - 1)], lane + QR[q])
    return cntA, (cntA + CH - 1) // CH, cntB, (cntB + CH - 1) // CH


def _scatter_range(t, table, zeros_hbm, tbl_hbm, out_hbm, gbase, size,
                   src2d, dst2d, nch, rows, didx, sem):
    pltpu.sync_copy(zeros_hbm.at[pl.ds(t * ZPT, ZPT)],
                    table.at[pl.ds(t * ZPT, ZPT)])
    plsc.subcore_barrier()

    def mbody(j, carry):
        for g in range(CH // 16):
            didx[pl.ds(g * 16, 16)] = dst2d[j, pl.ds(g * 16, 16)]
        pltpu.async_copy(tbl_hbm.at[src2d.at[j]], rows, sem).wait()
        pltpu.async_copy(rows, table.at[didx], sem, add=True).wait()
        return carry

    lax.fori_loop(0, nch, mbody, 0)
    plsc.subcore_barrier()
    _flush(table, out_hbm, t, gbase, size)
    plsc.subcore_barrier()


def _seg1_body(xi_hbm, xu_hbm, siu_hbm, diu_hbm, sui_hbm, dui_hbm, zeros_hbm,
               aggu_hbm, aggi_hbm, csrc_hbm, cdst_hbm, cnt_hbm,
               srcraw, dstraw, srcA, dstA, srcB, dstB, rows, cntv,
               didx, table, sem):
    c = lax.axis_index("c")
    t = lax.axis_index("s")
    w = c * NT + t
    base = c * HALF
    bufs = ((srcA, dstA), (srcB, dstB))
    lane = lax.iota(jnp.int32, 16)

    cA, nchA, cB, nchB = _compact2(t, base, siu_hbm, diu_hbm,
                                   srcraw, dstraw, bufs)
    _scatter_range(t, table, zeros_hbm, xi_hbm, aggu_hbm,
                   base, QR[0], srcA, dstA, nchA, rows, didx, sem)
    _scatter_range(t, table, zeros_hbm, xi_hbm, aggu_hbm,
                   base + QR[0], QR[1], srcB, dstB, nchB, rows, didx, sem)

    cA, nchA, cB, nchB = _compact2(t, base, sui_hbm, dui_hbm,
                                   srcraw, dstraw, bufs)
    pltpu.sync_copy(srcA, csrc_hbm.at[w * 2])
    pltpu.sync_copy(dstA, cdst_hbm.at[w * 2])
    pltpu.sync_copy(srcB, csrc_hbm.at[w * 2 + 1])
    pltpu.sync_copy(dstB, cdst_hbm.at[w * 2 + 1])
    cntv[...] = jnp.where(lane < 8, cA, cB)
    pltpu.sync_copy(cntv, cnt_hbm.at[w])
    _scatter_range(t, table, zeros_hbm, xu_hbm, aggi_hbm,
                   base, QR[0], srcA, dstA, nchA, rows, didx, sem)
    _scatter_range(t, table, zeros_hbm, xu_hbm, aggi_hbm,
                   base + QR[0], QR[1], srcB, dstB, nchB, rows, didx, sem)


def _seg2_body(hu0_hbm, hu1_hbm, csrc_hbm, cdst_hbm, cnt_hbm, zeros_hbm,
               agg20_hbm, agg21_hbm,
               srcA, dstA, srcB, dstB, rows, cntv, didx, table, sem):
    c = lax.axis_index("c")
    t = lax.axis_index("s")
    w = c * NT + t
    base = c * HALF
    pltpu.sync_copy(csrc_hbm.at[w * 2], srcA)
    pltpu.sync_copy(cdst_hbm.at[w * 2], dstA)
    pltpu.sync_copy(csrc_hbm.at[w * 2 + 1], srcB)
    pltpu.sync_copy(cdst_hbm.at[w * 2 + 1], dstB)
    pltpu.sync_copy(cnt_hbm.at[w], cntv)
    cv = cntv[...]
    cA = cv[0]
    cB = cv[8]
    nchA = (cA + CH - 1) // CH
    nchB = (cB + CH - 1) // CH
    for tbl_hbm, out_hbm in ((hu0_hbm, agg20_hbm), (hu1_hbm, agg21_hbm)):
        _scatter_range(t, table, zeros_hbm, tbl_hbm, out_hbm,
                       base, QR[0], srcA, dstA, nchA, rows, didx, sem)
        _scatter_range(t, table, zeros_hbm, tbl_hbm, out_hbm,
                       base + QR[0], QR[1], srcB, dstB, nchB, rows, didx, sem)


_SC_MESH = plsc.VectorSubcoreMesh(core_axis_name="c", subcore_axis_name="s")
_SC_PARAMS = pltpu.CompilerParams(needs_layout_passes=False,
                                  use_tc_tiling_on_sc=False)

_seg1 = pl.kernel(
    mesh=_SC_MESH,
    compiler_params=_SC_PARAMS,
    out_type=[jax.ShapeDtypeStruct((N, D), jnp.float32),
              jax.ShapeDtypeStruct((N, D), jnp.float32),
              jax.ShapeDtypeStruct((NC * NT * 2, MAXCH, CH), jnp.int32),
              jax.ShapeDtypeStruct((NC * NT * 2, MAXCH, CH), jnp.int32),
              jax.ShapeDtypeStruct((NC * NT, 16), jnp.int32)],
    scratch_types=[pltpu.VMEM((PIECE,), jnp.int32),
                   pltpu.VMEM((PIECE,), jnp.int32),
                   pltpu.VMEM((MAXCH, CH), jnp.int32),
                   pltpu.VMEM((MAXCH, CH), jnp.int32),
                   pltpu.VMEM((MAXCH, CH), jnp.int32),
                   pltpu.VMEM((MAXCH, CH), jnp.int32),
                   pltpu.VMEM((CH, D), jnp.float32),
                   pltpu.VMEM((16,), jnp.int32),
                   pltpu.VMEM((CH,), jnp.int32),
                   pltpu.VMEM_SHARED((TBLR, D), jnp.float32),
                   pltpu.SemaphoreType.DMA],
)(_seg1_body)

_seg2 = pl.kernel(
    mesh=_SC_MESH,
    compiler_params=_SC_PARAMS,
    out_type=[jax.ShapeDtypeStruct((N, D), jnp.float32),
              jax.ShapeDtypeStruct((N, D), jnp.float32)],
    scratch_types=[pltpu.VMEM((MAXCH, CH), jnp.int32),
                   pltpu.VMEM((MAXCH, CH), jnp.int32),
                   pltpu.VMEM((MAXCH, CH), jnp.int32),
                   pltpu.VMEM((MAXCH, CH), jnp.int32),
                   pltpu.VMEM((CH, D), jnp.float32),
                   pltpu.VMEM((16,), jnp.int32),
                   pltpu.VMEM((CH,), jnp.int32),
                   pltpu.VMEM_SHARED((TBLR, D), jnp.float32),
                   pltpu.SemaphoreType.DMA],
)(_seg2_body)


_RB = 2000
_RB2 = 1000


def _xi_body(x_ref, y_ref, emb_ref, o_ref):
    y = y_ref[...]
    w0 = emb_ref[0:1, :]
    w1 = emb_ref[1:2, :]
    add = jnp.where(y == 0, 1.0, 0.0) * w0 + jnp.where(y == 1, 1.0, 0.0) * w1
    o_ref[...] = x_ref[...] + add


def _xi_call(x_item, y2d, emb_weight):
    return pl.pallas_call(
        _xi_body,
        grid=(N // _RB,),
        in_specs=[pl.BlockSpec((_RB, D), lambda i: (i, 0)),
                  pl.BlockSpec((_RB, 1), lambda i: (i, 0)),
                  pl.BlockSpec((3, D), lambda i: (0, 0))],
        out_specs=pl.BlockSpec((_RB, D), lambda i: (i, 0)),
        out_shape=jax.ShapeDtypeStruct((N, D), jnp.float32),
    )(x_item, y2d, emb_weight)


def _mm1_body(aggu_ref, xu_ref, aggi_ref, xi_ref,
              wru_ref, wtu_ref, wri_ref, wti_ref,
              su_ref, bu_ref, si_ref, bi_ref,
              hu0_ref, hu1_ref, hi_ref):
    pre_u = (jnp.dot(aggu_ref[...], wru_ref[...],
                     preferred_element_type=jnp.float32)
             + jnp.dot(xu_ref[...], wtu_ref[...],
                       preferred_element_type=jnp.float32))
    hu = jnp.maximum(pre_u * su_ref[...] + bu_ref[...], 0.0)
    pre_i = (jnp.dot(aggi_ref[...], wri_ref[...],
                     preferred_element_type=jnp.float32)
             + jnp.dot(xi_ref[...], wti_ref[...],
                       preferred_element_type=jnp.float32))
    hi = jnp.maximum(pre_i * si_ref[...] + bi_ref[...], 0.0)
    hu0_ref[...] = hu[:, :D]
    hu1_ref[...] = hu[:, D:]
    hi_ref[...] = hi


def _mm1_call(agg_u, xu, agg_i, xi, wru, wtu, wri, wti, su, bu, si, bi):
    blk = lambda r, c: pl.BlockSpec((r, c), lambda i: (i, 0))
    full = lambda r, c: pl.BlockSpec((r, c), lambda i: (0, 0))
    return pl.pallas_call(
        _mm1_body,
        grid=(N // _RB2,),
        in_specs=[blk(_RB2, D), blk(_RB2, D), blk(_RB2, D), blk(_RB2, D),
                  full(D, H), full(D, H), full(D, H), full(D, H),
                  full(1, H), full(1, H), full(1, H), full(1, H)],
        out_specs=[blk(_RB2, D), blk(_RB2, D), blk(_RB2, H)],
        out_shape=[jax.ShapeDtypeStruct((N, D), jnp.float32),
                   jax.ShapeDtypeStruct((N, D), jnp.float32),
                   jax.ShapeDtypeStruct((N, H), jnp.float32)],
    )(agg_u, xu, agg_i, xi, wru, wtu, wri, wti, su, bu, si, bi)


def _fin_body(xi_ref, hi_ref, a0_ref, a1_ref,
              wa_ref, wb_ref, wc0_ref, wc1_ref, bias_ref, o_ref):
    acc = jnp.dot(xi_ref[...], wa_ref[...],
                  preferred_element_type=jnp.float32)
    acc += jnp.dot(hi_ref[...], wb_ref[...],
                   preferred_element_type=jnp.float32)
    acc += jnp.dot(a0_ref[...], wc0_ref[...],
                   preferred_element_type=jnp.float32)
    acc += jnp.dot(a1_ref[...], wc1_ref[...],
                   preferred_element_type=jnp.float32)
    o_ref[...] = acc + bias_ref[...]


def _fin_call(xi, hi, a0, a1, wa, wb, wc0, wc1, bias):
    blk = lambda r, c: pl.BlockSpec((r, c), lambda i: (i, 0))
    full = lambda r, c: pl.BlockSpec((r, c), lambda i: (0, 0))
    return pl.pallas_call(
        _fin_body,
        grid=(N // _RB2,),
        in_specs=[blk(_RB2, D), blk(_RB2, H), blk(_RB2, D), blk(_RB2, D),
                  full(D, 2), full(H, 2), full(D, 2), full(D, 2),
                  full(1, 2)],
        out_specs=blk(_RB2, 2),
        out_shape=jax.ShapeDtypeStruct((N, 2), jnp.float32),
    )(xi, hi, a0, a1, wa, wb, wc0, wc1, bias)


def kernel(x_item, x_user, edge_index_iu, edge_index_ui, y_emb, emb_weight,
           W_rel1_iu, b_rel1_iu, W_root1_iu, W_rel1_ui, b_rel1_ui, W_root1_ui,
           gamma_item, beta_item, gamma_user, beta_user,
           W_rel2_iu, b_rel2_iu, W_root2_iu, W_rel2_ui, b_rel2_ui, W_root2_ui,
           lin_W, lin_b):
    inv = 1.0 / jnp.sqrt(1.0 + 1e-5)
    su = (inv * gamma_user).reshape(1, H)
    bu = (b_rel1_iu * inv * gamma_user + beta_user).reshape(1, H)
    si = (inv * gamma_item).reshape(1, H)
    bi = (b_rel1_ui * inv * gamma_item + beta_item).reshape(1, H)
    wc = lin_W[D + H:]
    wa = lin_W[:D]
    wb = lin_W[D:D + H] + W_root2_ui @ wc
    wcf = W_rel2_ui @ wc
    bias = (lin_b + b_rel2_ui @ wc).reshape(1, 2)
    zeros_tbl = jnp.zeros((TBLR, D), jnp.float32)

    xi = _xi_call(x_item, y_emb.reshape(N, 1), emb_weight)
    agg_u, agg_i, csrc, cdst, cnts = _seg1(
        xi, x_user,
        edge_index_iu[0], edge_index_iu[1],
        edge_index_ui[0], edge_index_ui[1],
        zeros_tbl)
    hu0, hu1, hi = _mm1_call(agg_u, x_user, agg_i, xi,
                             W_rel1_iu, W_root1_iu, W_rel1_ui, W_root1_ui,
                             su, bu, si, bi)
    a0, a1 = _seg2(hu0, hu1, csrc, cdst, cnts, zeros_tbl)
    return _fin_call(xi, hi, a0, a1, wa, wb, wcf[:D], wcf[D:], bias)


# speedup vs baseline: 4.6131x; 1.2677x over previous
import jax
import jax.numpy as jnp
from jax import lax
from jax.experimental import pallas as pl
from jax.experimental.pallas import tpu as pltpu
from jax.experimental.pallas import tpu_sc as plsc

N = 10000
E = 160000
D = 256
H = 512

NC = 2
NT = 16
HALF = N // NC
QR = (2504, 2496)
TBLR = 2560
ZPT = TBLR // NT
EPT = E // NT
CH = 128
CHB = 7
MAXCH = 80
PIECE = 2000


def _flush(table, out_hbm, t, gbase, size):
    @pl.when(t < 12)
    def _():
        pltpu.sync_copy(table.at[pl.ds(t * 208, 208)],
                        out_hbm.at[pl.ds(gbase + t * 208, 208)])
    if size == 2504:
        @pl.when(t == 12)
        def _():
            pltpu.sync_copy(table.at[pl.ds(2496, 8)],
                            out_hbm.at[pl.ds(gbase + 2496, 8)])


def _compact2(t, base, srcv_hbm, dstv_hbm, srcraw, dstraw, bufs):
    (srcA, dstA), (srcB, dstB) = bufs
    baseB = base + QR[0]

    def piece(pi, carry):
        pltpu.sync_copy(srcv_hbm.at[pl.ds(t * EPT + pi * PIECE, PIECE)],
                        srcraw)
        pltpu.sync_copy(dstv_hbm.at[pl.ds(t * EPT + pi * PIECE, PIECE)],
                        dstraw)

        def cbody(j, carry):
            cntA, cntB = carry
            sv = srcraw[pl.ds(j * 16, 16)]
            dv = dstraw[pl.ds(j * 16, 16)]
            dA = dv - base
            mA = (dA >= 0) & (dA < QR[0])
            posA = plsc.cumsum(mA.astype(jnp.int32))
            iA = cntA + posA - 1
            plsc.store_scatter(srcA, [iA >> CHB, iA & (CH - 1)], sv, mask=mA)
            plsc.store_scatter(dstA, [iA >> CHB, iA & (CH - 1)], dA, mask=mA)
            dB = dv - baseB
            mB = (dB >= 0) & (dB < QR[1])
            posB = plsc.cumsum(mB.astype(jnp.int32))
            iB = cntB + posB - 1
            plsc.store_scatter(srcB, [iB >> CHB, iB & (CH - 1)], sv, mask=mB)
            plsc.store_scatter(dstB, [iB >> CHB, iB & (CH - 1)], dB, mask=mB)
            return (cntA + posA[15], cntB + posB[15])

        return lax.fori_loop(0, PIECE // 16, cbody, carry)

    cntA, cntB = lax.fori_loop(0, EPT // PIECE, piece,
                               (jnp.int32(0), jnp.int32(0)))
    lane = lax.iota(jnp.int32, 16)
    for (sbuf, dbuf, cnt, q) in ((srcA, dstA, cntA, 0), (srcB, dstB, cntB, 1)):
        for k in range(CH // 16):
            i2 = cnt + k * 16 + lane
            plsc.store_scatter(sbuf, [i2 >> CHB, i2 & (CH - 1)], lane + k * 16)
            plsc.store_scatter(dbuf, [i2 >> CHB, i2 & (CH - 1)], lane + QR[q])
    return cntA, (cntA + CH - 1) // CH, cntB, (cntB + CH - 1) // CH


SUB = CH // 2


def _scatter_range(t, table, zeros_hbm, tbl_hbm, out_hbm, gbase, size,
                   src2d, dst2d, nch, rows, didx, g0, g1, ssem):
    pltpu.sync_copy(zeros_hbm.at[pl.ds(t * ZPT, ZPT)],
                    table.at[pl.ds(t * ZPT, ZPT)])
    plsc.subcore_barrier()

    def gth(j, half, gsem):
        return pltpu.make_async_copy(
            tbl_hbm.at[src2d.at[j, pl.ds(half * SUB, SUB)]],
            rows.at[pl.ds(half * SUB, SUB)], gsem)

    def scat(j, half):
        for g in range(SUB // 16):
            didx[pl.ds(g * 16, 16)] = dst2d[j, pl.ds(half * SUB + g * 16, 16)]
        pltpu.async_copy(rows.at[pl.ds(half * SUB, SUB)],
                         table.at[didx], ssem, add=True).wait()

    @pl.when(nch > 0)
    def _():
        gth(0, 0, g0).start()

    def mbody(j, carry):
        gth(j, 1, g1).start()
        gth(j, 0, g0).wait()
        scat(j, 0)

        @pl.when(j + 1 < nch)
        def _():
            gth(j + 1, 0, g0).start()

        gth(j, 1, g1).wait()
        scat(j, 1)
        return carry

    lax.fori_loop(0, nch, mbody, 0)
    plsc.subcore_barrier()
    _flush(table, out_hbm, t, gbase, size)
    plsc.subcore_barrier()


def _seg1_body(xi_hbm, xu_hbm, siu_hbm, diu_hbm, sui_hbm, dui_hbm, zeros_hbm,
               aggu_hbm, aggi_hbm, csrc_hbm, cdst_hbm, cnt_hbm,
               srcraw, dstraw, srcA, dstA, srcB, dstB, rows, cntv,
               didx, table, g0, g1, ssem):
    c = lax.axis_index("c")
    t = lax.axis_index("s")
    w = c * NT + t
    base = c * HALF
    bufs = ((srcA, dstA), (srcB, dstB))
    lane = lax.iota(jnp.int32, 16)

    cA, nchA, cB, nchB = _compact2(t, base, siu_hbm, diu_hbm,
                                   srcraw, dstraw, bufs)
    _scatter_range(t, table, zeros_hbm, xi_hbm, aggu_hbm,
                   base, QR[0], srcA, dstA, nchA, rows, didx, g0, g1, ssem)
    _scatter_range(t, table, zeros_hbm, xi_hbm, aggu_hbm,
                   base + QR[0], QR[1], srcB, dstB, nchB, rows, didx, g0, g1, ssem)

    cA, nchA, cB, nchB = _compact2(t, base, sui_hbm, dui_hbm,
                                   srcraw, dstraw, bufs)
    pltpu.sync_copy(srcA, csrc_hbm.at[w * 2])
    pltpu.sync_copy(dstA, cdst_hbm.at[w * 2])
    pltpu.sync_copy(srcB, csrc_hbm.at[w * 2 + 1])
    pltpu.sync_copy(dstB, cdst_hbm.at[w * 2 + 1])
    cntv[...] = jnp.where(lane < 8, cA, cB)
    pltpu.sync_copy(cntv, cnt_hbm.at[w])
    _scatter_range(t, table, zeros_hbm, xu_hbm, aggi_hbm,
                   base, QR[0], srcA, dstA, nchA, rows, didx, g0, g1, ssem)
    _scatter_range(t, table, zeros_hbm, xu_hbm, aggi_hbm,
                   base + QR[0], QR[1], srcB, dstB, nchB, rows, didx, g0, g1, ssem)


def _seg2_body(hu0_hbm, hu1_hbm, csrc_hbm, cdst_hbm, cnt_hbm, zeros_hbm,
               agg20_hbm, agg21_hbm,
               srcA, dstA, srcB, dstB, rows, cntv, didx, table,
               g0, g1, ssem):
    c = lax.axis_index("c")
    t = lax.axis_index("s")
    w = c * NT + t
    base = c * HALF
    pltpu.sync_copy(csrc_hbm.at[w * 2], srcA)
    pltpu.sync_copy(cdst_hbm.at[w * 2], dstA)
    pltpu.sync_copy(csrc_hbm.at[w * 2 + 1], srcB)
    pltpu.sync_copy(cdst_hbm.at[w * 2 + 1], dstB)
    pltpu.sync_copy(cnt_hbm.at[w], cntv)
    cv = cntv[...]
    cA = cv[0]
    cB = cv[8]
    nchA = (cA + CH - 1) // CH
    nchB = (cB + CH - 1) // CH
    for tbl_hbm, out_hbm in ((hu0_hbm, agg20_hbm), (hu1_hbm, agg21_hbm)):
        _scatter_range(t, table, zeros_hbm, tbl_hbm, out_hbm,
                       base, QR[0], srcA, dstA, nchA, rows, didx, g0, g1, ssem)
        _scatter_range(t, table, zeros_hbm, tbl_hbm, out_hbm,
                       base + QR[0], QR[1], srcB, dstB, nchB, rows, didx, g0, g1, ssem)


_SC_MESH = plsc.VectorSubcoreMesh(core_axis_name="c", subcore_axis_name="s")
_SC_PARAMS = pltpu.CompilerParams(needs_layout_passes=False,
                                  use_tc_tiling_on_sc=False)

_seg1 = pl.kernel(
    mesh=_SC_MESH,
    compiler_params=_SC_PARAMS,
    out_type=[jax.ShapeDtypeStruct((N, D), jnp.float32),
              jax.ShapeDtypeStruct((N, D), jnp.float32),
              jax.ShapeDtypeStruct((NC * NT * 2, MAXCH, CH), jnp.int32),
              jax.ShapeDtypeStruct((NC * NT * 2, MAXCH, CH), jnp.int32),
              jax.ShapeDtypeStruct((NC * NT, 16), jnp.int32)],
    scratch_types=[pltpu.VMEM((PIECE,), jnp.int32),
                   pltpu.VMEM((PIECE,), jnp.int32),
                   pltpu.VMEM((MAXCH, CH), jnp.int32),
                   pltpu.VMEM((MAXCH, CH), jnp.int32),
                   pltpu.VMEM((MAXCH, CH), jnp.int32),
                   pltpu.VMEM((MAXCH, CH), jnp.int32),
                   pltpu.VMEM((CH, D), jnp.float32),
                   pltpu.VMEM((16,), jnp.int32),
                   pltpu.VMEM((SUB,), jnp.int32),
                   pltpu.VMEM_SHARED((TBLR, D), jnp.float32),
                   pltpu.SemaphoreType.DMA,
                   pltpu.SemaphoreType.DMA,
                   pltpu.SemaphoreType.DMA],
)(_seg1_body)

_seg2 = pl.kernel(
    mesh=_SC_MESH,
    compiler_params=_SC_PARAMS,
    out_type=[jax.ShapeDtypeStruct((N, D), jnp.float32),
              jax.ShapeDtypeStruct((N, D), jnp.float32)],
    scratch_types=[pltpu.VMEM((MAXCH, CH), jnp.int32),
                   pltpu.VMEM((MAXCH, CH), jnp.int32),
                   pltpu.VMEM((MAXCH, CH), jnp.int32),
                   pltpu.VMEM((MAXCH, CH), jnp.int32),
                   pltpu.VMEM((CH, D), jnp.float32),
                   pltpu.VMEM((16,), jnp.int32),
                   pltpu.VMEM((SUB,), jnp.int32),
                   pltpu.VMEM_SHARED((TBLR, D), jnp.float32),
                   pltpu.SemaphoreType.DMA,
                   pltpu.SemaphoreType.DMA,
                   pltpu.SemaphoreType.DMA],
)(_seg2_body)


_RB = 2000
_RB2 = 1000


def _xi_body(x_ref, y_ref, emb_ref, o_ref):
    y = y_ref[...]
    w0 = emb_ref[0:1, :]
    w1 = emb_ref[1:2, :]
    add = jnp.where(y == 0, 1.0, 0.0) * w0 + jnp.where(y == 1, 1.0, 0.0) * w1
    o_ref[...] = x_ref[...] + add


def _xi_call(x_item, y2d, emb_weight):
    return pl.pallas_call(
        _xi_body,
        grid=(N // _RB,),
        in_specs=[pl.BlockSpec((_RB, D), lambda i: (i, 0)),
                  pl.BlockSpec((_RB, 1), lambda i: (i, 0)),
                  pl.BlockSpec((3, D), lambda i: (0, 0))],
        out_specs=pl.BlockSpec((_RB, D), lambda i: (i, 0)),
        out_shape=jax.ShapeDtypeStruct((N, D), jnp.float32),
    )(x_item, y2d, emb_weight)


def _mm1_body(aggu_ref, xu_ref, aggi_ref, xi_ref,
              wru_ref, wtu_ref, wri_ref, wti_ref,
              su_ref, bu_ref, si_ref, bi_ref,
              hu0_ref, hu1_ref, hi_ref):
    pre_u = (jnp.dot(aggu_ref[...], wru_ref[...],
                     preferred_element_type=jnp.float32)
             + jnp.dot(xu_ref[...], wtu_ref[...],
                       preferred_element_type=jnp.float32))
    hu = jnp.maximum(pre_u * su_ref[...] + bu_ref[...], 0.0)
    pre_i = (jnp.dot(aggi_ref[...], wri_ref[...],
                     preferred_element_type=jnp.float32)
             + jnp.dot(xi_ref[...], wti_ref[...],
                       preferred_element_type=jnp.float32))
    hi = jnp.maximum(pre_i * si_ref[...] + bi_ref[...], 0.0)
    hu0_ref[...] = hu[:, :D]
    hu1_ref[...] = hu[:, D:]
    hi_ref[...] = hi


def _mm1_call(agg_u, xu, agg_i, xi, wru, wtu, wri, wti, su, bu, si, bi):
    blk = lambda r, c: pl.BlockSpec((r, c), lambda i: (i, 0))
    full = lambda r, c: pl.BlockSpec((r, c), lambda i: (0, 0))
    return pl.pallas_call(
        _mm1_body,
        grid=(N // _RB2,),
        in_specs=[blk(_RB2, D), blk(_RB2, D), blk(_RB2, D), blk(_RB2, D),
                  full(D, H), full(D, H), full(D, H), full(D, H),
                  full(1, H), full(1, H), full(1, H), full(1, H)],
        out_specs=[blk(_RB2, D), blk(_RB2, D), blk(_RB2, H)],
        out_shape=[jax.ShapeDtypeStruct((N, D), jnp.float32),
                   jax.ShapeDtypeStruct((N, D), jnp.float32),
                   jax.ShapeDtypeStruct((N, H), jnp.float32)],
    )(agg_u, xu, agg_i, xi, wru, wtu, wri, wti, su, bu, si, bi)


def _fin_body(xi_ref, hi_ref, a0_ref, a1_ref,
              wa_ref, wb_ref, wc0_ref, wc1_ref, bias_ref, o_ref):
    acc = jnp.dot(xi_ref[...], wa_ref[...],
                  preferred_element_type=jnp.float32)
    acc += jnp.dot(hi_ref[...], wb_ref[...],
                   preferred_element_type=jnp.float32)
    acc += jnp.dot(a0_ref[...], wc0_ref[...],
                   preferred_element_type=jnp.float32)
    acc += jnp.dot(a1_ref[...], wc1_ref[...],
                   preferred_element_type=jnp.float32)
    o_ref[...] = acc + bias_ref[...]


def _fin_call(xi, hi, a0, a1, wa, wb, wc0, wc1, bias):
    blk = lambda r, c: pl.BlockSpec((r, c), lambda i: (i, 0))
    full = lambda r, c: pl.BlockSpec((r, c), lambda i: (0, 0))
    return pl.pallas_call(
        _fin_body,
        grid=(N // _RB2,),
        in_specs=[blk(_RB2, D), blk(_RB2, H), blk(_RB2, D), blk(_RB2, D),
                  full(D, 2), full(H, 2), full(D, 2), full(D, 2),
                  full(1, 2)],
        out_specs=blk(_RB2, 2),
        out_shape=jax.ShapeDtypeStruct((N, 2), jnp.float32),
    )(xi, hi, a0, a1, wa, wb, wc0, wc1, bias)


def kernel(x_item, x_user, edge_index_iu, edge_index_ui, y_emb, emb_weight,
           W_rel1_iu, b_rel1_iu, W_root1_iu, W_rel1_ui, b_rel1_ui, W_root1_ui,
           gamma_item, beta_item, gamma_user, beta_user,
           W_rel2_iu, b_rel2_iu, W_root2_iu, W_rel2_ui, b_rel2_ui, W_root2_ui,
           lin_W, lin_b):
    inv = 1.0 / jnp.sqrt(1.0 + 1e-5)
    su = (inv * gamma_user).reshape(1, H)
    bu = (b_rel1_iu * inv * gamma_user + beta_user).reshape(1, H)
    si = (inv * gamma_item).reshape(1, H)
    bi = (b_rel1_ui * inv * gamma_item + beta_item).reshape(1, H)
    wc = lin_W[D + H:]
    wa = lin_W[:D]
    wb = lin_W[D:D + H] + W_root2_ui @ wc
    wcf = W_rel2_ui @ wc
    bias = (lin_b + b_rel2_ui @ wc).reshape(1, 2)
    zeros_tbl = jnp.zeros((TBLR, D), jnp.float32)

    xi = _xi_call(x_item, y_emb.reshape(N, 1), emb_weight)
    agg_u, agg_i, csrc, cdst, cnts = _seg1(
        xi, x_user,
        edge_index_iu[0], edge_index_iu[1],
        edge_index_ui[0], edge_index_ui[1],
        zeros_tbl)
    hu0, hu1, hi = _mm1_call(agg_u, x_user, agg_i, xi,
                             W_rel1_iu, W_root1_iu, W_rel1_ui, W_root1_ui,
                             su, bu, si, bi)
    a0, a1 = _seg2(hu0, hu1, csrc, cdst, cnts, zeros_tbl)
    return _fin_call(xi, hi, a0, a1, wa, wb, wcf[:D], wcf[D:], bias)


# speedup vs baseline: 4.8446x; 1.0502x over previous
import jax
import jax.numpy as jnp
from jax import lax
from jax.experimental import pallas as pl
from jax.experimental.pallas import tpu as pltpu
from jax.experimental.pallas import tpu_sc as plsc

N = 10000
E = 160000
D = 256
H = 512

NC = 2
NT = 16
HALF = N // NC
QR = (2504, 2496)
TBLR = 2560
ZPT = TBLR // NT
EPT = E // NT
CH = 128
CHB = 7
MAXCH = 80
PIECE = 2000


def _flush(table, out_hbm, t, gbase, size):
    @pl.when(t < 12)
    def _():
        pltpu.sync_copy(table.at[pl.ds(t * 208, 208)],
                        out_hbm.at[pl.ds(gbase + t * 208, 208)])
    if size == 2504:
        @pl.when(t == 12)
        def _():
            pltpu.sync_copy(table.at[pl.ds(2496, 8)],
                            out_hbm.at[pl.ds(gbase + 2496, 8)])


def _compact2(t, base, srcv_hbm, dstv_hbm, srcraw, dstraw, bufs):
    (srcA, dstA), (srcB, dstB) = bufs
    baseB = base + QR[0]

    def piece(pi, carry):
        pltpu.sync_copy(srcv_hbm.at[pl.ds(t * EPT + pi * PIECE, PIECE)],
                        srcraw)
        pltpu.sync_copy(dstv_hbm.at[pl.ds(t * EPT + pi * PIECE, PIECE)],
                        dstraw)

        def cbody(j, carry):
            cntA, cntB = carry
            sv = srcraw[pl.ds(j * 16, 16)]
            dv = dstraw[pl.ds(j * 16, 16)]
            dA = dv - base
            mA = (dA >= 0) & (dA < QR[0])
            posA = plsc.cumsum(mA.astype(jnp.int32))
            iA = cntA + posA - 1
            plsc.store_scatter(srcA, [iA >> CHB, iA & (CH - 1)], sv, mask=mA)
            plsc.store_scatter(dstA, [iA >> CHB, iA & (CH - 1)], dA, mask=mA)
            dB = dv - baseB
            mB = (dB >= 0) & (dB < QR[1])
            posB = plsc.cumsum(mB.astype(jnp.int32))
            iB = cntB + posB - 1
            plsc.store_scatter(srcB, [iB >> CHB, iB & (CH - 1)], sv, mask=mB)
            plsc.store_scatter(dstB, [iB >> CHB, iB & (CH - 1)], dB, mask=mB)
            return (cntA + posA[15], cntB + posB[15])

        return lax.fori_loop(0, PIECE // 16, cbody, carry)

    cntA, cntB = lax.fori_loop(0, EPT // PIECE, piece,
                               (jnp.int32(0), jnp.int32(0)))
    lane = lax.iota(jnp.int32, 16)
    for (sbuf, dbuf, cnt, q) in ((srcA, dstA, cntA, 0), (srcB, dstB, cntB, 1)):
        for k in range(CH // 16):
            i2 = cnt + k * 16 + lane
            plsc.store_scatter(sbuf, [i2 >> CHB, i2 & (CH - 1)], lane + k * 16)
            plsc.store_scatter(dbuf, [i2 >> CHB, i2 & (CH - 1)], lane + QR[q])
    return cntA, (cntA + CH - 1) // CH, cntB, (cntB + CH - 1) // CH


SUB = CH // 2


def _scatter_range(t, table, zeros_hbm, tbl_hbm, out_hbm, gbase, size,
                   src2d, dst2d, nch, rows, didx, g0, g1, ssem):
    pltpu.sync_copy(zeros_hbm.at[pl.ds(t * ZPT, ZPT)],
                    table.at[pl.ds(t * ZPT, ZPT)])
    plsc.subcore_barrier()

    def gth(j, half, gsem):
        return pltpu.make_async_copy(
            tbl_hbm.at[src2d.at[j, pl.ds(half * SUB, SUB)]],
            rows.at[pl.ds(half * SUB, SUB)], gsem)

    def scat(j, half):
        for g in range(SUB // 16):
            didx[pl.ds(g * 16, 16)] = dst2d[j, pl.ds(half * SUB + g * 16, 16)]
        pltpu.async_copy(rows.at[pl.ds(half * SUB, SUB)],
                         table.at[didx], ssem, add=True).wait()

    @pl.when(nch > 0)
    def _():
        gth(0, 0, g0).start()

    def mbody(j, carry):
        gth(j, 1, g1).start()
        gth(j, 0, g0).wait()
        scat(j, 0)

        @pl.when(j + 1 < nch)
        def _():
            gth(j + 1, 0, g0).start()

        gth(j, 1, g1).wait()
        scat(j, 1)
        return carry

    lax.fori_loop(0, nch, mbody, 0)
    plsc.subcore_barrier()
    _flush(table, out_hbm, t, gbase, size)
    plsc.subcore_barrier()


def _seg1a_body(xi_hbm, siu_hbm, diu_hbm, zeros_hbm,
                aggu_hbm,
                srcraw, dstraw, srcA, dstA, srcB, dstB, rows, cntv,
                didx, table, g0, g1, ssem):
    c = lax.axis_index("c")
    t = lax.axis_index("s")
    base = c * HALF
    bufs = ((srcA, dstA), (srcB, dstB))

    cA, nchA, cB, nchB = _compact2(t, base, siu_hbm, diu_hbm,
                                   srcraw, dstraw, bufs)
    _scatter_range(t, table, zeros_hbm, xi_hbm, aggu_hbm,
                   base, QR[0], srcA, dstA, nchA, rows, didx, g0, g1, ssem)
    _scatter_range(t, table, zeros_hbm, xi_hbm, aggu_hbm,
                   base + QR[0], QR[1], srcB, dstB, nchB, rows, didx, g0, g1, ssem)


def _seg1b_body(xu_hbm, sui_hbm, dui_hbm, zeros_hbm,
                aggi_hbm, csrc_hbm, cdst_hbm, cnt_hbm,
                srcraw, dstraw, srcA, dstA, srcB, dstB, rows, cntv,
                didx, table, g0, g1, ssem):
    c = lax.axis_index("c")
    t = lax.axis_index("s")
    w = c * NT + t
    base = c * HALF
    bufs = ((srcA, dstA), (srcB, dstB))
    lane = lax.iota(jnp.int32, 16)

    cA, nchA, cB, nchB = _compact2(t, base, sui_hbm, dui_hbm,
                                   srcraw, dstraw, bufs)
    pltpu.sync_copy(srcA, csrc_hbm.at[w * 2])
    pltpu.sync_copy(dstA, cdst_hbm.at[w * 2])
    pltpu.sync_copy(srcB, csrc_hbm.at[w * 2 + 1])
    pltpu.sync_copy(dstB, cdst_hbm.at[w * 2 + 1])
    cntv[...] = jnp.where(lane < 8, cA, cB)
    pltpu.sync_copy(cntv, cnt_hbm.at[w])
    _scatter_range(t, table, zeros_hbm, xu_hbm, aggi_hbm,
                   base, QR[0], srcA, dstA, nchA, rows, didx, g0, g1, ssem)
    _scatter_range(t, table, zeros_hbm, xu_hbm, aggi_hbm,
                   base + QR[0], QR[1], srcB, dstB, nchB, rows, didx, g0, g1, ssem)


def _seg2_body(hu0_hbm, hu1_hbm, csrc_hbm, cdst_hbm, cnt_hbm, zeros_hbm,
               agg20_hbm, agg21_hbm,
               srcA, dstA, srcB, dstB, rows, cntv, didx, table,
               g0, g1, ssem):
    c = lax.axis_index("c")
    t = lax.axis_index("s")
    w = c * NT + t
    base = c * HALF
    pltpu.sync_copy(csrc_hbm.at[w * 2], srcA)
    pltpu.sync_copy(cdst_hbm.at[w * 2], dstA)
    pltpu.sync_copy(csrc_hbm.at[w * 2 + 1], srcB)
    pltpu.sync_copy(cdst_hbm.at[w * 2 + 1], dstB)
    pltpu.sync_copy(cnt_hbm.at[w], cntv)
    cv = cntv[...]
    cA = cv[0]
    cB = cv[8]
    nchA = (cA + CH - 1) // CH
    nchB = (cB + CH - 1) // CH
    for tbl_hbm, out_hbm in ((hu0_hbm, agg20_hbm), (hu1_hbm, agg21_hbm)):
        _scatter_range(t, table, zeros_hbm, tbl_hbm, out_hbm,
                       base, QR[0], srcA, dstA, nchA, rows, didx, g0, g1, ssem)
        _scatter_range(t, table, zeros_hbm, tbl_hbm, out_hbm,
                       base + QR[0], QR[1], srcB, dstB, nchB, rows, didx, g0, g1, ssem)


_SC_MESH = plsc.VectorSubcoreMesh(core_axis_name="c", subcore_axis_name="s")
_SC_PARAMS = pltpu.CompilerParams(needs_layout_passes=False,
                                  use_tc_tiling_on_sc=False)

_SEG_SCRATCH = [pltpu.VMEM((PIECE,), jnp.int32),
                pltpu.VMEM((PIECE,), jnp.int32),
                pltpu.VMEM((MAXCH, CH), jnp.int32),
                pltpu.VMEM((MAXCH, CH), jnp.int32),
                pltpu.VMEM((MAXCH, CH), jnp.int32),
                pltpu.VMEM((MAXCH, CH), jnp.int32),
                pltpu.VMEM((CH, D), jnp.float32),
                pltpu.VMEM((16,), jnp.int32),
                pltpu.VMEM((SUB,), jnp.int32),
                pltpu.VMEM_SHARED((TBLR, D), jnp.float32),
                pltpu.SemaphoreType.DMA,
                pltpu.SemaphoreType.DMA,
                pltpu.SemaphoreType.DMA]

_seg1a = pl.kernel(
    mesh=_SC_MESH,
    compiler_params=_SC_PARAMS,
    out_type=[jax.ShapeDtypeStruct((N, D), jnp.float32)],
    scratch_types=_SEG_SCRATCH,
)(_seg1a_body)

_seg1b = pl.kernel(
    mesh=_SC_MESH,
    compiler_params=_SC_PARAMS,
    out_type=[jax.ShapeDtypeStruct((N, D), jnp.float32),
              jax.ShapeDtypeStruct((NC * NT * 2, MAXCH, CH), jnp.int32),
              jax.ShapeDtypeStruct((NC * NT * 2, MAXCH, CH), jnp.int32),
              jax.ShapeDtypeStruct((NC * NT, 16), jnp.int32)],
    scratch_types=_SEG_SCRATCH,
)(_seg1b_body)

_seg2 = pl.kernel(
    mesh=_SC_MESH,
    compiler_params=_SC_PARAMS,
    out_type=[jax.ShapeDtypeStruct((N, D), jnp.float32),
              jax.ShapeDtypeStruct((N, D), jnp.float32)],
    scratch_types=[pltpu.VMEM((MAXCH, CH), jnp.int32),
                   pltpu.VMEM((MAXCH, CH), jnp.int32),
                   pltpu.VMEM((MAXCH, CH), jnp.int32),
                   pltpu.VMEM((MAXCH, CH), jnp.int32),
                   pltpu.VMEM((CH, D), jnp.float32),
                   pltpu.VMEM((16,), jnp.int32),
                   pltpu.VMEM((SUB,), jnp.int32),
                   pltpu.VMEM_SHARED((TBLR, D), jnp.float32),
                   pltpu.SemaphoreType.DMA,
                   pltpu.SemaphoreType.DMA,
                   pltpu.SemaphoreType.DMA],
)(_seg2_body)


_RB = 2000
_RB2 = 1000


def _xi_body(x_ref, y_ref, emb_ref, o_ref):
    y = y_ref[...]
    w0 = emb_ref[0:1, :]
    w1 = emb_ref[1:2, :]
    add = jnp.where(y == 0, 1.0, 0.0) * w0 + jnp.where(y == 1, 1.0, 0.0) * w1
    o_ref[...] = x_ref[...] + add


def _xi_call(x_item, y2d, emb_weight):
    return pl.pallas_call(
        _xi_body,
        grid=(N // _RB,),
        in_specs=[pl.BlockSpec((_RB, D), lambda i: (i, 0)),
                  pl.BlockSpec((_RB, 1), lambda i: (i, 0)),
                  pl.BlockSpec((3, D), lambda i: (0, 0))],
        out_specs=pl.BlockSpec((_RB, D), lambda i: (i, 0)),
        out_shape=jax.ShapeDtypeStruct((N, D), jnp.float32),
    )(x_item, y2d, emb_weight)


def _mm_u_body(aggu_ref, xu_ref, wru_ref, wtu_ref, su_ref, bu_ref,
               hu0_ref, hu1_ref):
    pre_u = (jnp.dot(aggu_ref[...], wru_ref[...],
                     preferred_element_type=jnp.float32)
             + jnp.dot(xu_ref[...], wtu_ref[...],
                       preferred_element_type=jnp.float32))
    hu = jnp.maximum(pre_u * su_ref[...] + bu_ref[...], 0.0)
    hu0_ref[...] = hu[:, :D]
    hu1_ref[...] = hu[:, D:]


def _mm_u_call(agg_u, xu, wru, wtu, su, bu):
    blk = lambda r, c: pl.BlockSpec((r, c), lambda i: (i, 0))
    full = lambda r, c: pl.BlockSpec((r, c), lambda i: (0, 0))
    return pl.pallas_call(
        _mm_u_body,
        grid=(N // _RB2,),
        in_specs=[blk(_RB2, D), blk(_RB2, D),
                  full(D, H), full(D, H), full(1, H), full(1, H)],
        out_specs=[blk(_RB2, D), blk(_RB2, D)],
        out_shape=[jax.ShapeDtypeStruct((N, D), jnp.float32),
                   jax.ShapeDtypeStruct((N, D), jnp.float32)],
    )(agg_u, xu, wru, wtu, su, bu)


def _mm_i_body(aggi_ref, xi_ref, wri_ref, wti_ref, si_ref, bi_ref, hi_ref):
    pre_i = (jnp.dot(aggi_ref[...], wri_ref[...],
                     preferred_element_type=jnp.float32)
             + jnp.dot(xi_ref[...], wti_ref[...],
                       preferred_element_type=jnp.float32))
    hi_ref[...] = jnp.maximum(pre_i * si_ref[...] + bi_ref[...], 0.0)


def _mm_i_call(agg_i, xi, wri, wti, si, bi):
    blk = lambda r, c: pl.BlockSpec((r, c), lambda i: (i, 0))
    full = lambda r, c: pl.BlockSpec((r, c), lambda i: (0, 0))
    return pl.pallas_call(
        _mm_i_body,
        grid=(N // _RB2,),
        in_specs=[blk(_RB2, D), blk(_RB2, D),
                  full(D, H), full(D, H), full(1, H), full(1, H)],
        out_specs=blk(_RB2, H),
        out_shape=jax.ShapeDtypeStruct((N, H), jnp.float32),
    )(agg_i, xi, wri, wti, si, bi)


def _fin_body(xi_ref, hi_ref, a0_ref, a1_ref,
              wa_ref, wb_ref, wc0_ref, wc1_ref, bias_ref, o_ref):
    acc = jnp.dot(xi_ref[...], wa_ref[...],
                  preferred_element_type=jnp.float32)
    acc += jnp.dot(hi_ref[...], wb_ref[...],
                   preferred_element_type=jnp.float32)
    acc += jnp.dot(a0_ref[...], wc0_ref[...],
                   preferred_element_type=jnp.float32)
    acc += jnp.dot(a1_ref[...], wc1_ref[...],
                   preferred_element_type=jnp.float32)
    o_ref[...] = acc + bias_ref[...]


def _fin_call(xi, hi, a0, a1, wa, wb, wc0, wc1, bias):
    blk = lambda r, c: pl.BlockSpec((r, c), lambda i: (i, 0))
    full = lambda r, c: pl.BlockSpec((r, c), lambda i: (0, 0))
    return pl.pallas_call(
        _fin_body,
        grid=(N // _RB2,),
        in_specs=[blk(_RB2, D), blk(_RB2, H), blk(_RB2, D), blk(_RB2, D),
                  full(D, 2), full(H, 2), full(D, 2), full(D, 2),
                  full(1, 2)],
        out_specs=blk(_RB2, 2),
        out_shape=jax.ShapeDtypeStruct((N, 2), jnp.float32),
    )(xi, hi, a0, a1, wa, wb, wc0, wc1, bias)


def kernel(x_item, x_user, edge_index_iu, edge_index_ui, y_emb, emb_weight,
           W_rel1_iu, b_rel1_iu, W_root1_iu, W_rel1_ui, b_rel1_ui, W_root1_ui,
           gamma_item, beta_item, gamma_user, beta_user,
           W_rel2_iu, b_rel2_iu, W_root2_iu, W_rel2_ui, b_rel2_ui, W_root2_ui,
           lin_W, lin_b):
    inv = 1.0 / jnp.sqrt(1.0 + 1e-5)
    su = (inv * gamma_user).reshape(1, H)
    bu = (b_rel1_iu * inv * gamma_user + beta_user).reshape(1, H)
    si = (inv * gamma_item).reshape(1, H)
    bi = (b_rel1_ui * inv * gamma_item + beta_item).reshape(1, H)
    wc = lin_W[D + H:]
    wa = lin_W[:D]
    wb = lin_W[D:D + H] + W_root2_ui @ wc
    wcf = W_rel2_ui @ wc
    bias = (lin_b + b_rel2_ui @ wc).reshape(1, 2)
    zeros_tbl = jnp.zeros((TBLR, D), jnp.float32)

    xi = _xi_call(x_item, y_emb.reshape(N, 1), emb_weight)
    agg_u, = _seg1a(xi, edge_index_iu[0], edge_index_iu[1], zeros_tbl)
    agg_i, csrc, cdst, cnts = _seg1b(
        x_user, edge_index_ui[0], edge_index_ui[1], zeros_tbl)
    hu0, hu1 = _mm_u_call(agg_u, x_user, W_rel1_iu, W_root1_iu, su, bu)
    hi = _mm_i_call(agg_i, xi, W_rel1_ui, W_root1_ui, si, bi)
    a0, a1 = _seg2(hu0, hu1, csrc, cdst, cnts, zeros_tbl)
    return _fin_call(xi, hi, a0, a1, wa, wb, wcf[:D], wcf[D:], bias)


# speedup vs baseline: 5.7536x; 1.1876x over previous
import jax
import jax.numpy as jnp
from jax import lax
from jax.experimental import pallas as pl
from jax.experimental.pallas import tpu as pltpu
from jax.experimental.pallas import tpu_sc as plsc

N = 10000
E = 160000
D = 256
H = 512

NC = 2
NT = 16
HALF = N // NC
QR = (2504, 2496)
TBLR = 2560
ZPT = TBLR // NT
EPT = E // NT
CH = 128
CHB = 7
MAXCH = 80
PIECE = 2000


def _flush(table, out_hbm, t, gbase, size):
    @pl.when(t < 12)
    def _():
        pltpu.sync_copy(table.at[pl.ds(t * 208, 208)],
                        out_hbm.at[pl.ds(gbase + t * 208, 208)])
    if size == 2504:
        @pl.when(t == 12)
        def _():
            pltpu.sync_copy(table.at[pl.ds(2496, 8)],
                            out_hbm.at[pl.ds(gbase + 2496, 8)])


def _compact2(t, base, srcv_hbm, dstv_hbm, srcraw, dstraw, bufs):
    (srcA, dstA), (srcB, dstB) = bufs
    baseB = base + QR[0]

    def piece(pi, carry):
        pltpu.sync_copy(srcv_hbm.at[pl.ds(t * EPT + pi * PIECE, PIECE)],
                        srcraw)
        pltpu.sync_copy(dstv_hbm.at[pl.ds(t * EPT + pi * PIECE, PIECE)],
                        dstraw)

        def cbody(j, carry):
            cntA, cntB = carry
            sv = srcraw[pl.ds(j * 16, 16)]
            dv = dstraw[pl.ds(j * 16, 16)]
            dA = dv - base
            mA = (dA >= 0) & (dA < QR[0])
            posA = plsc.cumsum(mA.astype(jnp.int32))
            iA = cntA + posA - 1
            plsc.store_scatter(srcA, [iA >> CHB, iA & (CH - 1)], sv, mask=mA)
            plsc.store_scatter(dstA, [iA >> CHB, iA & (CH - 1)], dA, mask=mA)
            dB = dv - baseB
            mB = (dB >= 0) & (dB < QR[1])
            posB = plsc.cumsum(mB.astype(jnp.int32))
            iB = cntB + posB - 1
            plsc.store_scatter(srcB, [iB >> CHB, iB & (CH - 1)], sv, mask=mB)
            plsc.store_scatter(dstB, [iB >> CHB, iB & (CH - 1)], dB, mask=mB)
            return (cntA + posA[15], cntB + posB[15])

        return lax.fori_loop(0, PIECE // 16, cbody, carry)

    cntA, cntB = lax.fori_loop(0, EPT // PIECE, piece,
                               (jnp.int32(0), jnp.int32(0)))
    lane = lax.iota(jnp.int32, 16)
    for (sbuf, dbuf, cnt, q) in ((srcA, dstA, cntA, 0), (srcB, dstB, cntB, 1)):
        for k in range(CH // 16):
            i2 = cnt + k * 16 + lane
            plsc.store_scatter(sbuf, [i2 >> CHB, i2 & (CH - 1)], lane + k * 16)
            plsc.store_scatter(dbuf, [i2 >> CHB, i2 & (CH - 1)], lane + QR[q])
    return cntA, (cntA + CH - 1) // CH, cntB, (cntB + CH - 1) // CH


SUB = CH // 2


def _scatter_range(t, table, zeros_hbm, tbl_hbm, out_hbm, gbase, size,
                   src2d, dst2d, nch, rows, didx, g0, g1, ssem):
    pltpu.sync_copy(zeros_hbm.at[pl.ds(t * ZPT, ZPT)],
                    table.at[pl.ds(t * ZPT, ZPT)])
    plsc.subcore_barrier()

    def gth(j, half, gsem):
        return pltpu.make_async_copy(
            tbl_hbm.at[src2d.at[j, pl.ds(half * SUB, SUB)]],
            rows.at[pl.ds(half * SUB, SUB)], gsem)

    def scat(j, half):
        for g in range(SUB // 16):
            didx[pl.ds(g * 16, 16)] = dst2d[j, pl.ds(half * SUB + g * 16, 16)]
        pltpu.async_copy(rows.at[pl.ds(half * SUB, SUB)],
                         table.at[didx], ssem, add=True).wait()

    @pl.when(nch > 0)
    def _():
        gth(0, 0, g0).start()

    def mbody(j, carry):
        gth(j, 1, g1).start()
        gth(j, 0, g0).wait()
        scat(j, 0)

        @pl.when(j + 1 < nch)
        def _():
            gth(j + 1, 0, g0).start()

        gth(j, 1, g1).wait()
        scat(j, 1)
        return carry

    lax.fori_loop(0, nch, mbody, 0)
    plsc.subcore_barrier()
    _flush(table, out_hbm, t, gbase, size)
    plsc.subcore_barrier()


def _seg1a_body(xi_hbm, siu_hbm, diu_hbm, zeros_hbm,
                aggu_hbm,
                srcraw, dstraw, srcA, dstA, srcB, dstB, rows, cntv,
                didx, table, g0, g1, ssem):
    c = lax.axis_index("c")
    t = lax.axis_index("s")
    base = c * HALF
    bufs = ((srcA, dstA), (srcB, dstB))

    cA, nchA, cB, nchB = _compact2(t, base, siu_hbm, diu_hbm,
                                   srcraw, dstraw, bufs)
    _scatter_range(t, table, zeros_hbm, xi_hbm, aggu_hbm,
                   base, QR[0], srcA, dstA, nchA, rows, didx, g0, g1, ssem)
    _scatter_range(t, table, zeros_hbm, xi_hbm, aggu_hbm,
                   base + QR[0], QR[1], srcB, dstB, nchB, rows, didx, g0, g1, ssem)


def _seg1b_body(xu_hbm, sui_hbm, dui_hbm, zeros_hbm,
                aggi_hbm, csrc_hbm, cdst_hbm, cnt_hbm,
                srcraw, dstraw, srcA, dstA, srcB, dstB, rows, cntv,
                didx, table, g0, g1, ssem):
    c = lax.axis_index("c")
    t = lax.axis_index("s")
    w = c * NT + t
    base = c * HALF
    bufs = ((srcA, dstA), (srcB, dstB))
    lane = lax.iota(jnp.int32, 16)

    cA, nchA, cB, nchB = _compact2(t, base, sui_hbm, dui_hbm,
                                   srcraw, dstraw, bufs)
    pltpu.sync_copy(srcA, csrc_hbm.at[w * 2])
    pltpu.sync_copy(dstA, cdst_hbm.at[w * 2])
    pltpu.sync_copy(srcB, csrc_hbm.at[w * 2 + 1])
    pltpu.sync_copy(dstB, cdst_hbm.at[w * 2 + 1])
    cntv[...] = jnp.where(lane < 8, cA, cB)
    pltpu.sync_copy(cntv, cnt_hbm.at[w])
    _scatter_range(t, table, zeros_hbm, xu_hbm, aggi_hbm,
                   base, QR[0], srcA, dstA, nchA, rows, didx, g0, g1, ssem)
    _scatter_range(t, table, zeros_hbm, xu_hbm, aggi_hbm,
                   base + QR[0], QR[1], srcB, dstB, nchB, rows, didx, g0, g1, ssem)


def _seg2_body(hu0_hbm, hu1_hbm, csrc_hbm, cdst_hbm, cnt_hbm, zeros_hbm,
               agg20_hbm, agg21_hbm,
               srcA, dstA, srcB, dstB, rows, cntv, didx, table,
               g0, g1, ssem):
    c = lax.axis_index("c")
    t = lax.axis_index("s")
    w = c * NT + t
    base = c * HALF
    pltpu.sync_copy(csrc_hbm.at[w * 2], srcA)
    pltpu.sync_copy(cdst_hbm.at[w * 2], dstA)
    pltpu.sync_copy(csrc_hbm.at[w * 2 + 1], srcB)
    pltpu.sync_copy(cdst_hbm.at[w * 2 + 1], dstB)
    pltpu.sync_copy(cnt_hbm.at[w], cntv)
    cv = cntv[...]
    cA = cv[0]
    cB = cv[8]
    nchA = (cA + CH - 1) // CH
    nchB = (cB + CH - 1) // CH
    for tbl_hbm, out_hbm in ((hu0_hbm, agg20_hbm), (hu1_hbm, agg21_hbm)):
        _scatter_range(t, table, zeros_hbm, tbl_hbm, out_hbm,
                       base, QR[0], srcA, dstA, nchA, rows, didx, g0, g1, ssem)
        _scatter_range(t, table, zeros_hbm, tbl_hbm, out_hbm,
                       base + QR[0], QR[1], srcB, dstB, nchB, rows, didx, g0, g1, ssem)


_SC_MESH = plsc.VectorSubcoreMesh(core_axis_name="c", subcore_axis_name="s")
_SC_PARAMS = pltpu.CompilerParams(needs_layout_passes=False,
                                  use_tc_tiling_on_sc=False)

_SEG_SCRATCH = [pltpu.VMEM((PIECE,), jnp.int32),
                pltpu.VMEM((PIECE,), jnp.int32),
                pltpu.VMEM((MAXCH, CH), jnp.int32),
                pltpu.VMEM((MAXCH, CH), jnp.int32),
                pltpu.VMEM((MAXCH, CH), jnp.int32),
                pltpu.VMEM((MAXCH, CH), jnp.int32),
                pltpu.VMEM((CH, D), jnp.float32),
                pltpu.VMEM((16,), jnp.int32),
                pltpu.VMEM((SUB,), jnp.int32),
                pltpu.VMEM_SHARED((TBLR, D), jnp.float32),
                pltpu.SemaphoreType.DMA,
                pltpu.SemaphoreType.DMA,
                pltpu.SemaphoreType.DMA]

_seg1a = pl.kernel(
    mesh=_SC_MESH,
    compiler_params=_SC_PARAMS,
    out_type=[jax.ShapeDtypeStruct((N, D), jnp.float32)],
    scratch_types=_SEG_SCRATCH,
)(_seg1a_body)

_seg1b = pl.kernel(
    mesh=_SC_MESH,
    compiler_params=_SC_PARAMS,
    out_type=[jax.ShapeDtypeStruct((N, D), jnp.float32),
              jax.ShapeDtypeStruct((NC * NT * 2, MAXCH, CH), jnp.int32),
              jax.ShapeDtypeStruct((NC * NT * 2, MAXCH, CH), jnp.int32),
              jax.ShapeDtypeStruct((NC * NT, 16), jnp.int32)],
    scratch_types=_SEG_SCRATCH,
)(_seg1b_body)

_seg2 = pl.kernel(
    mesh=_SC_MESH,
    compiler_params=_SC_PARAMS,
    out_type=[jax.ShapeDtypeStruct((N, D), jnp.bfloat16),
              jax.ShapeDtypeStruct((N, D), jnp.bfloat16)],
    scratch_types=[pltpu.VMEM((MAXCH, CH), jnp.int32),
                   pltpu.VMEM((MAXCH, CH), jnp.int32),
                   pltpu.VMEM((MAXCH, CH), jnp.int32),
                   pltpu.VMEM((MAXCH, CH), jnp.int32),
                   pltpu.VMEM((CH, D), jnp.bfloat16),
                   pltpu.VMEM((16,), jnp.int32),
                   pltpu.VMEM((SUB,), jnp.int32),
                   pltpu.VMEM_SHARED((TBLR, D), jnp.bfloat16),
                   pltpu.SemaphoreType.DMA,
                   pltpu.SemaphoreType.DMA,
                   pltpu.SemaphoreType.DMA],
)(_seg2_body)


_RB = 2000
_RB2 = 1000


def _xi_body(x_ref, y_ref, emb_ref, o_ref):
    y = y_ref[...]
    w0 = emb_ref[0:1, :]
    w1 = emb_ref[1:2, :]
    add = jnp.where(y == 0, 1.0, 0.0) * w0 + jnp.where(y == 1, 1.0, 0.0) * w1
    o_ref[...] = x_ref[...] + add


def _xi_call(x_item, y2d, emb_weight):
    return pl.pallas_call(
        _xi_body,
        grid=(N // _RB,),
        in_specs=[pl.BlockSpec((_RB, D), lambda i: (i, 0)),
                  pl.BlockSpec((_RB, 1), lambda i: (i, 0)),
                  pl.BlockSpec((3, D), lambda i: (0, 0))],
        out_specs=pl.BlockSpec((_RB, D), lambda i: (i, 0)),
        out_shape=jax.ShapeDtypeStruct((N, D), jnp.float32),
    )(x_item, y2d, emb_weight)


def _mm_u_body(aggu_ref, xu_ref, wru_ref, wtu_ref, su_ref, bu_ref,
               hu0_ref, hu1_ref):
    pre_u = (jnp.dot(aggu_ref[...], wru_ref[...],
                     preferred_element_type=jnp.float32)
             + jnp.dot(xu_ref[...], wtu_ref[...],
                       preferred_element_type=jnp.float32))
    hu = jnp.maximum(pre_u * su_ref[...] + bu_ref[...], 0.0)
    hu0_ref[...] = hu[:, :D].astype(jnp.bfloat16)
    hu1_ref[...] = hu[:, D:].astype(jnp.bfloat16)


def _mm_u_call(agg_u, xu, wru, wtu, su, bu):
    blk = lambda r, c: pl.BlockSpec((r, c), lambda i: (i, 0))
    full = lambda r, c: pl.BlockSpec((r, c), lambda i: (0, 0))
    return pl.pallas_call(
        _mm_u_body,
        grid=(N // _RB,),
        in_specs=[blk(_RB, D), blk(_RB, D),
                  full(D, H), full(D, H), full(1, H), full(1, H)],
        out_specs=[blk(_RB, D), blk(_RB, D)],
        out_shape=[jax.ShapeDtypeStruct((N, D), jnp.bfloat16),
                   jax.ShapeDtypeStruct((N, D), jnp.bfloat16)],
    )(agg_u, xu, wru, wtu, su, bu)


def _mm_i_body(aggi_ref, xi_ref, wri_ref, wti_ref, si_ref, bi_ref, hi_ref):
    pre_i = (jnp.dot(aggi_ref[...], wri_ref[...],
                     preferred_element_type=jnp.float32)
             + jnp.dot(xi_ref[...], wti_ref[...],
                       preferred_element_type=jnp.float32))
    hi_ref[...] = jnp.maximum(pre_i * si_ref[...] + bi_ref[...], 0.0)


def _mm_i_call(agg_i, xi, wri, wti, si, bi):
    blk = lambda r, c: pl.BlockSpec((r, c), lambda i: (i, 0))
    full = lambda r, c: pl.BlockSpec((r, c), lambda i: (0, 0))
    return pl.pallas_call(
        _mm_i_body,
        grid=(N // _RB2,),
        in_specs=[blk(_RB2, D), blk(_RB2, D),
                  full(D, H), full(D, H), full(1, H), full(1, H)],
        out_specs=blk(_RB2, H),
        out_shape=jax.ShapeDtypeStruct((N, H), jnp.float32),
    )(agg_i, xi, wri, wti, si, bi)


def _pp_body(xi_ref, hi_ref, wa_ref, wb_ref, bias_ref, o_ref):
    acc = jnp.dot(xi_ref[...], wa_ref[...],
                  preferred_element_type=jnp.float32)
    acc += jnp.dot(hi_ref[...], wb_ref[...],
                   preferred_element_type=jnp.float32)
    o_ref[...] = acc + bias_ref[...]


def _pp_call(xi, hi, wa, wb, bias):
    blk = lambda r, c: pl.BlockSpec((r, c), lambda i: (i, 0))
    full = lambda r, c: pl.BlockSpec((r, c), lambda i: (0, 0))
    return pl.pallas_call(
        _pp_body,
        grid=(N // _RB2,),
        in_specs=[blk(_RB2, D), blk(_RB2, H),
                  full(D, 2), full(H, 2), full(1, 2)],
        out_specs=blk(_RB2, 2),
        out_shape=jax.ShapeDtypeStruct((N, 2), jnp.float32),
    )(xi, hi, wa, wb, bias)


def _fin_body(pp_ref, a0_ref, a1_ref, wc0_ref, wc1_ref, o_ref):
    acc = jnp.dot(a0_ref[...].astype(jnp.float32), wc0_ref[...],
                  preferred_element_type=jnp.float32)
    acc += jnp.dot(a1_ref[...].astype(jnp.float32), wc1_ref[...],
                   preferred_element_type=jnp.float32)
    o_ref[...] = acc + pp_ref[...]


def _fin_call(pp, a0, a1, wc0, wc1):
    blk = lambda r, c: pl.BlockSpec((r, c), lambda i: (i, 0))
    full = lambda r, c: pl.BlockSpec((r, c), lambda i: (0, 0))
    return pl.pallas_call(
        _fin_body,
        grid=(N // _RB,),
        in_specs=[blk(_RB, 2), blk(_RB, D), blk(_RB, D),
                  full(D, 2), full(D, 2)],
        out_specs=blk(_RB, 2),
        out_shape=jax.ShapeDtypeStruct((N, 2), jnp.float32),
    )(pp, a0, a1, wc0, wc1)


def kernel(x_item, x_user, edge_index_iu, edge_index_ui, y_emb, emb_weight,
           W_rel1_iu, b_rel1_iu, W_root1_iu, W_rel1_ui, b_rel1_ui, W_root1_ui,
           gamma_item, beta_item, gamma_user, beta_user,
           W_rel2_iu, b_rel2_iu, W_root2_iu, W_rel2_ui, b_rel2_ui, W_root2_ui,
           lin_W, lin_b):
    inv = 1.0 / jnp.sqrt(1.0 + 1e-5)
    su = (inv * gamma_user).reshape(1, H)
    bu = (b_rel1_iu * inv * gamma_user + beta_user).reshape(1, H)
    si = (inv * gamma_item).reshape(1, H)
    bi = (b_rel1_ui * inv * gamma_item + beta_item).reshape(1, H)
    wc = lin_W[D + H:]
    wa = lin_W[:D]
    wb = lin_W[D:D + H] + W_root2_ui @ wc
    wcf = W_rel2_ui @ wc
    bias = (lin_b + b_rel2_ui @ wc).reshape(1, 2)
    zeros_tbl = jnp.zeros((TBLR, D), jnp.float32)
    zeros_bf16 = jnp.zeros((TBLR, D), jnp.bfloat16)

    xi = _xi_call(x_item, y_emb.reshape(N, 1), emb_weight)
    agg_u, = _seg1a(xi, edge_index_iu[0], edge_index_iu[1], zeros_tbl)
    agg_i, csrc, cdst, cnts = _seg1b(
        x_user, edge_index_ui[0], edge_index_ui[1], zeros_tbl)
    hu0, hu1 = _mm_u_call(agg_u, x_user, W_rel1_iu, W_root1_iu, su, bu)
    hi = _mm_i_call(agg_i, xi, W_rel1_ui, W_root1_ui, si, bi)
    pp = _pp_call(xi, hi, wa, wb, bias)
    a0, a1 = _seg2(hu0, hu1, csrc, cdst, cnts, zeros_bf16)
    return _fin_call(pp, a0, a1, wcf[:D], wcf[D:])


# speedup vs baseline: 6.7511x; 1.1734x over previous
import jax
import jax.numpy as jnp
from jax import lax
from jax.experimental import pallas as pl
from jax.experimental.pallas import tpu as pltpu
from jax.experimental.pallas import tpu_sc as plsc

N = 10000
E = 160000
D = 256
H = 512

NC = 2
NT = 16
HALF = N // NC
QR = (2504, 2496)
TBLR = 2560
ZPT = TBLR // NT
EPT = E // NT
CH = 128
CHB = 7
MAXCH = 80
PIECE = 2000


def _flush(table, out_hbm, t, gbase, size):
    @pl.when(t < 12)
    def _():
        pltpu.sync_copy(table.at[pl.ds(t * 208, 208)],
                        out_hbm.at[pl.ds(gbase + t * 208, 208)])
    if size == 2504:
        @pl.when(t == 12)
        def _():
            pltpu.sync_copy(table.at[pl.ds(2496, 8)],
                            out_hbm.at[pl.ds(gbase + 2496, 8)])


def _compact2(t, base, srcv_hbm, dstv_hbm, srcraw, dstraw, bufs):
    (srcA, dstA), (srcB, dstB) = bufs
    baseB = base + QR[0]

    def piece(pi, carry):
        pltpu.sync_copy(srcv_hbm.at[pl.ds(t * EPT + pi * PIECE, PIECE)],
                        srcraw)
        pltpu.sync_copy(dstv_hbm.at[pl.ds(t * EPT + pi * PIECE, PIECE)],
                        dstraw)

        def cbody(j, carry):
            cntA, cntB = carry
            sv = srcraw[pl.ds(j * 16, 16)]
            dv = dstraw[pl.ds(j * 16, 16)]
            dA = dv - base
            mA = (dA >= 0) & (dA < QR[0])
            posA = plsc.cumsum(mA.astype(jnp.int32))
            iA = cntA + posA - 1
            plsc.store_scatter(srcA, [iA >> CHB, iA & (CH - 1)], sv, mask=mA)
            plsc.store_scatter(dstA, [iA >> CHB, iA & (CH - 1)], dA, mask=mA)
            dB = dv - baseB
            mB = (dB >= 0) & (dB < QR[1])
            posB = plsc.cumsum(mB.astype(jnp.int32))
            iB = cntB + posB - 1
            plsc.store_scatter(srcB, [iB >> CHB, iB & (CH - 1)], sv, mask=mB)
            plsc.store_scatter(dstB, [iB >> CHB, iB & (CH - 1)], dB, mask=mB)
            return (cntA + posA[15], cntB + posB[15])

        return lax.fori_loop(0, PIECE // 16, cbody, carry)

    cntA, cntB = lax.fori_loop(0, EPT // PIECE, piece,
                               (jnp.int32(0), jnp.int32(0)))
    lane = lax.iota(jnp.int32, 16)
    for (sbuf, dbuf, cnt, q) in ((srcA, dstA, cntA, 0), (srcB, dstB, cntB, 1)):
        for k in range(CH // 16):
            i2 = cnt + k * 16 + lane
            plsc.store_scatter(sbuf, [i2 >> CHB, i2 & (CH - 1)], lane + k * 16)
            plsc.store_scatter(dbuf, [i2 >> CHB, i2 & (CH - 1)], lane + QR[q])
    return cntA, (cntA + CH - 1) // CH, cntB, (cntB + CH - 1) // CH


SUB = CH // 2


def _scatter_range(t, table, zeros_hbm, tbl_hbm, out_hbm, gbase, size,
                   src2d, dst2d, nch, rows, didx, g0, g1, ssem):
    pltpu.sync_copy(zeros_hbm.at[pl.ds(t * ZPT, ZPT)],
                    table.at[pl.ds(t * ZPT, ZPT)])
    plsc.subcore_barrier()

    def gth(j, half, gsem):
        return pltpu.make_async_copy(
            tbl_hbm.at[src2d.at[j, pl.ds(half * SUB, SUB)]],
            rows.at[pl.ds(half * SUB, SUB)], gsem)

    def scat(j, half):
        for g in range(SUB // 16):
            didx[pl.ds(g * 16, 16)] = dst2d[j, pl.ds(half * SUB + g * 16, 16)]
        pltpu.async_copy(rows.at[pl.ds(half * SUB, SUB)],
                         table.at[didx], ssem, add=True).wait()

    @pl.when(nch > 0)
    def _():
        gth(0, 0, g0).start()

    def mbody(j, carry):
        gth(j, 1, g1).start()
        gth(j, 0, g0).wait()
        scat(j, 0)

        @pl.when(j + 1 < nch)
        def _():
            gth(j + 1, 0, g0).start()

        gth(j, 1, g1).wait()
        scat(j, 1)
        return carry

    lax.fori_loop(0, nch, mbody, 0)
    plsc.subcore_barrier()
    _flush(table, out_hbm, t, gbase, size)
    plsc.subcore_barrier()


def _seg1a_body(xi_hbm, siu_hbm, diu_hbm, zeros_hbm,
                aggu_hbm,
                srcraw, dstraw, srcA, dstA, srcB, dstB, rows, cntv,
                didx, table, g0, g1, ssem):
    c = lax.axis_index("c")
    t = lax.axis_index("s")
    base = c * HALF
    bufs = ((srcA, dstA), (srcB, dstB))

    cA, nchA, cB, nchB = _compact2(t, base, siu_hbm, diu_hbm,
                                   srcraw, dstraw, bufs)
    _scatter_range(t, table, zeros_hbm, xi_hbm, aggu_hbm,
                   base, QR[0], srcA, dstA, nchA, rows, didx, g0, g1, ssem)
    _scatter_range(t, table, zeros_hbm, xi_hbm, aggu_hbm,
                   base + QR[0], QR[1], srcB, dstB, nchB, rows, didx, g0, g1, ssem)


def _seg1b_body(xu_hbm, sui_hbm, dui_hbm, zeros_hbm,
                aggi_hbm, csrc_hbm, cdst_hbm, cnt_hbm,
                srcraw, dstraw, srcA, dstA, srcB, dstB, rows, cntv,
                didx, table, g0, g1, ssem):
    c = lax.axis_index("c")
    t = lax.axis_index("s")
    w = c * NT + t
    base = c * HALF
    bufs = ((srcA, dstA), (srcB, dstB))
    lane = lax.iota(jnp.int32, 16)

    cA, nchA, cB, nchB = _compact2(t, base, sui_hbm, dui_hbm,
                                   srcraw, dstraw, bufs)
    pltpu.sync_copy(srcA, csrc_hbm.at[w * 2])
    pltpu.sync_copy(dstA, cdst_hbm.at[w * 2])
    pltpu.sync_copy(srcB, csrc_hbm.at[w * 2 + 1])
    pltpu.sync_copy(dstB, cdst_hbm.at[w * 2 + 1])
    cntv[...] = jnp.where(lane < 8, cA, cB)
    pltpu.sync_copy(cntv, cnt_hbm.at[w])
    _scatter_range(t, table, zeros_hbm, xu_hbm, aggi_hbm,
                   base, QR[0], srcA, dstA, nchA, rows, didx, g0, g1, ssem)
    _scatter_range(t, table, zeros_hbm, xu_hbm, aggi_hbm,
                   base + QR[0], QR[1], srcB, dstB, nchB, rows, didx, g0, g1, ssem)


def _seg2_body(hu0_hbm, hu1_hbm, csrc_hbm, cdst_hbm, cnt_hbm, zeros_hbm,
               agg20_hbm, agg21_hbm,
               srcA, dstA, srcB, dstB, rows, cntv, didx, table,
               g0, g1, ssem):
    c = lax.axis_index("c")
    t = lax.axis_index("s")
    w = c * NT + t
    base = c * HALF
    pltpu.sync_copy(csrc_hbm.at[w * 2], srcA)
    pltpu.sync_copy(cdst_hbm.at[w * 2], dstA)
    pltpu.sync_copy(csrc_hbm.at[w * 2 + 1], srcB)
    pltpu.sync_copy(cdst_hbm.at[w * 2 + 1], dstB)
    pltpu.sync_copy(cnt_hbm.at[w], cntv)
    cv = cntv[...]
    cA = cv[0]
    cB = cv[8]
    nchA = (cA + CH - 1) // CH
    nchB = (cB + CH - 1) // CH
    for tbl_hbm, out_hbm in ((hu0_hbm, agg20_hbm), (hu1_hbm, agg21_hbm)):
        _scatter_range(t, table, zeros_hbm, tbl_hbm, out_hbm,
                       base, QR[0], srcA, dstA, nchA, rows, didx, g0, g1, ssem)
        _scatter_range(t, table, zeros_hbm, tbl_hbm, out_hbm,
                       base + QR[0], QR[1], srcB, dstB, nchB, rows, didx, g0, g1, ssem)


_SC_MESH = plsc.VectorSubcoreMesh(core_axis_name="c", subcore_axis_name="s")
_SC_PARAMS = pltpu.CompilerParams(needs_layout_passes=False,
                                  use_tc_tiling_on_sc=False)

_SEG_SCRATCH = [pltpu.VMEM((PIECE,), jnp.int32),
                pltpu.VMEM((PIECE,), jnp.int32),
                pltpu.VMEM((MAXCH, CH), jnp.int32),
                pltpu.VMEM((MAXCH, CH), jnp.int32),
                pltpu.VMEM((MAXCH, CH), jnp.int32),
                pltpu.VMEM((MAXCH, CH), jnp.int32),
                pltpu.VMEM((CH, D), jnp.bfloat16),
                pltpu.VMEM((16,), jnp.int32),
                pltpu.VMEM((SUB,), jnp.int32),
                pltpu.VMEM_SHARED((TBLR, D), jnp.bfloat16),
                pltpu.SemaphoreType.DMA,
                pltpu.SemaphoreType.DMA,
                pltpu.SemaphoreType.DMA]

_seg1a = pl.kernel(
    mesh=_SC_MESH,
    compiler_params=_SC_PARAMS,
    out_type=[jax.ShapeDtypeStruct((N, D), jnp.bfloat16)],
    scratch_types=_SEG_SCRATCH,
)(_seg1a_body)

_seg1b = pl.kernel(
    mesh=_SC_MESH,
    compiler_params=_SC_PARAMS,
    out_type=[jax.ShapeDtypeStruct((N, D), jnp.bfloat16),
              jax.ShapeDtypeStruct((NC * NT * 2, MAXCH, CH), jnp.int32),
              jax.ShapeDtypeStruct((NC * NT * 2, MAXCH, CH), jnp.int32),
              jax.ShapeDtypeStruct((NC * NT, 16), jnp.int32)],
    scratch_types=_SEG_SCRATCH,
)(_seg1b_body)

_seg2 = pl.kernel(
    mesh=_SC_MESH,
    compiler_params=_SC_PARAMS,
    out_type=[jax.ShapeDtypeStruct((N, D), jnp.bfloat16),
              jax.ShapeDtypeStruct((N, D), jnp.bfloat16)],
    scratch_types=[pltpu.VMEM((MAXCH, CH), jnp.int32),
                   pltpu.VMEM((MAXCH, CH), jnp.int32),
                   pltpu.VMEM((MAXCH, CH), jnp.int32),
                   pltpu.VMEM((MAXCH, CH), jnp.int32),
                   pltpu.VMEM((CH, D), jnp.bfloat16),
                   pltpu.VMEM((16,), jnp.int32),
                   pltpu.VMEM((SUB,), jnp.int32),
                   pltpu.VMEM_SHARED((TBLR, D), jnp.bfloat16),
                   pltpu.SemaphoreType.DMA,
                   pltpu.SemaphoreType.DMA,
                   pltpu.SemaphoreType.DMA],
)(_seg2_body)


_RB = 2000
_RB2 = 1000


def _xi_body(x_ref, y_ref, emb_ref, o_ref, o16_ref):
    y = y_ref[...]
    w0 = emb_ref[0:1, :]
    w1 = emb_ref[1:2, :]
    add = jnp.where(y == 0, 1.0, 0.0) * w0 + jnp.where(y == 1, 1.0, 0.0) * w1
    xi = x_ref[...] + add
    o_ref[...] = xi
    o16_ref[...] = xi.astype(jnp.bfloat16)


def _xi_call(x_item, y2d, emb_weight):
    return pl.pallas_call(
        _xi_body,
        grid=(N // _RB,),
        in_specs=[pl.BlockSpec((_RB, D), lambda i: (i, 0)),
                  pl.BlockSpec((_RB, 1), lambda i: (i, 0)),
                  pl.BlockSpec((3, D), lambda i: (0, 0))],
        out_specs=[pl.BlockSpec((_RB, D), lambda i: (i, 0)),
                   pl.BlockSpec((_RB, D), lambda i: (i, 0))],
        out_shape=[jax.ShapeDtypeStruct((N, D), jnp.float32),
                   jax.ShapeDtypeStruct((N, D), jnp.bfloat16)],
    )(x_item, y2d, emb_weight)


def _mm_u_body(aggu_ref, xu_ref, wru_ref, wtu_ref, su_ref, bu_ref,
               hu0_ref, hu1_ref):
    pre_u = (jnp.dot(aggu_ref[...].astype(jnp.float32), wru_ref[...],
                     preferred_element_type=jnp.float32)
             + jnp.dot(xu_ref[...], wtu_ref[...],
                       preferred_element_type=jnp.float32))
    hu = jnp.maximum(pre_u * su_ref[...] + bu_ref[...], 0.0)
    hu0_ref[...] = hu[:, :D].astype(jnp.bfloat16)
    hu1_ref[...] = hu[:, D:].astype(jnp.bfloat16)


def _mm_u_call(agg_u, xu, wru, wtu, su, bu):
    blk = lambda r, c: pl.BlockSpec((r, c), lambda i: (i, 0))
    full = lambda r, c: pl.BlockSpec((r, c), lambda i: (0, 0))
    return pl.pallas_call(
        _mm_u_body,
        grid=(N // _RB,),
        in_specs=[blk(_RB, D), blk(_RB, D),
                  full(D, H), full(D, H), full(1, H), full(1, H)],
        out_specs=[blk(_RB, D), blk(_RB, D)],
        out_shape=[jax.ShapeDtypeStruct((N, D), jnp.bfloat16),
                   jax.ShapeDtypeStruct((N, D), jnp.bfloat16)],
    )(agg_u, xu, wru, wtu, su, bu)


def _mm_i_body(aggi_ref, xi_ref, wri_ref, wti_ref, si_ref, bi_ref, hi_ref):
    pre_i = (jnp.dot(aggi_ref[...].astype(jnp.float32), wri_ref[...],
                     preferred_element_type=jnp.float32)
             + jnp.dot(xi_ref[...], wti_ref[...],
                       preferred_element_type=jnp.float32))
    hi_ref[...] = jnp.maximum(pre_i * si_ref[...] + bi_ref[...], 0.0)


def _mm_i_call(agg_i, xi, wri, wti, si, bi):
    blk = lambda r, c: pl.BlockSpec((r, c), lambda i: (i, 0))
    full = lambda r, c: pl.BlockSpec((r, c), lambda i: (0, 0))
    return pl.pallas_call(
        _mm_i_body,
        grid=(N // _RB,),
        in_specs=[blk(_RB, D), blk(_RB, D),
                  full(D, H), full(D, H), full(1, H), full(1, H)],
        out_specs=blk(_RB, H),
        out_shape=jax.ShapeDtypeStruct((N, H), jnp.float32),
    )(agg_i, xi, wri, wti, si, bi)


def _pp_body(xi_ref, hi_ref, wa_ref, wb_ref, bias_ref, o_ref):
    acc = jnp.dot(xi_ref[...], wa_ref[...],
                  preferred_element_type=jnp.float32)
    acc += jnp.dot(hi_ref[...], wb_ref[...],
                   preferred_element_type=jnp.float32)
    o_ref[...] = acc + bias_ref[...]


def _pp_call(xi, hi, wa, wb, bias):
    blk = lambda r, c: pl.BlockSpec((r, c), lambda i: (i, 0))
    full = lambda r, c: pl.BlockSpec((r, c), lambda i: (0, 0))
    return pl.pallas_call(
        _pp_body,
        grid=(N // _RB2,),
        in_specs=[blk(_RB2, D), blk(_RB2, H),
                  full(D, 2), full(H, 2), full(1, 2)],
        out_specs=blk(_RB2, 2),
        out_shape=jax.ShapeDtypeStruct((N, 2), jnp.float32),
    )(xi, hi, wa, wb, bias)


def _fin_body(pp_ref, a0_ref, a1_ref, wc0_ref, wc1_ref, o_ref):
    acc = jnp.dot(a0_ref[...].astype(jnp.float32), wc0_ref[...],
                  preferred_element_type=jnp.float32)
    acc += jnp.dot(a1_ref[...].astype(jnp.float32), wc1_ref[...],
                   preferred_element_type=jnp.float32)
    o_ref[...] = acc + pp_ref[...]


def _fin_call(pp, a0, a1, wc0, wc1):
    blk = lambda r, c: pl.BlockSpec((r, c), lambda i: (i, 0))
    full = lambda r, c: pl.BlockSpec((r, c), lambda i: (0, 0))
    return pl.pallas_call(
        _fin_body,
        grid=(N // _RB,),
        in_specs=[blk(_RB, 2), blk(_RB, D), blk(_RB, D),
                  full(D, 2), full(D, 2)],
        out_specs=blk(_RB, 2),
        out_shape=jax.ShapeDtypeStruct((N, 2), jnp.float32),
    )(pp, a0, a1, wc0, wc1)


def kernel(x_item, x_user, edge_index_iu, edge_index_ui, y_emb, emb_weight,
           W_rel1_iu, b_rel1_iu, W_root1_iu, W_rel1_ui, b_rel1_ui, W_root1_ui,
           gamma_item, beta_item, gamma_user, beta_user,
           W_rel2_iu, b_rel2_iu, W_root2_iu, W_rel2_ui, b_rel2_ui, W_root2_ui,
           lin_W, lin_b):
    inv = 1.0 / jnp.sqrt(1.0 + 1e-5)
    su = (inv * gamma_user).reshape(1, H)
    bu = (b_rel1_iu * inv * gamma_user + beta_user).reshape(1, H)
    si = (inv * gamma_item).reshape(1, H)
    bi = (b_rel1_ui * inv * gamma_item + beta_item).reshape(1, H)
    wc = lin_W[D + H:]
    wa = lin_W[:D]
    wb = lin_W[D:D + H] + W_root2_ui @ wc
    wcf = W_rel2_ui @ wc
    bias = (lin_b + b_rel2_ui @ wc).reshape(1, 2)
    zeros_bf16 = jnp.zeros((TBLR, D), jnp.bfloat16)

    xi, xi16 = _xi_call(x_item, y_emb.reshape(N, 1), emb_weight)
    xu16 = x_user.astype(jnp.bfloat16)
    agg_u, = _seg1a(xi16, edge_index_iu[0], edge_index_iu[1], zeros_bf16)
    agg_i, csrc, cdst, cnts = _seg1b(
        xu16, edge_index_ui[0], edge_index_ui[1], zeros_bf16)
    hu0, hu1 = _mm_u_call(agg_u, x_user, W_rel1_iu, W_root1_iu, su, bu)
    hi = _mm_i_call(agg_i, xi, W_rel1_ui, W_root1_ui, si, bi)
    pp = _pp_call(xi, hi, wa, wb, bias)
    a0, a1 = _seg2(hu0, hu1, csrc, cdst, cnts, zeros_bf16)
    return _fin_call(pp, a0, a1, wcf[:D], wcf[D:])


# speedup vs baseline: 6.8908x; 1.0207x over previous
import jax
import jax.numpy as jnp
from jax import lax
from jax.experimental import pallas as pl
from jax.experimental.pallas import tpu as pltpu
from jax.experimental.pallas import tpu_sc as plsc

N = 10000
E = 160000
D = 256
H = 512

NC = 2
NT = 16
HALF = N // NC
QR = (2504, 2496)
TBLR = 2560
ZPT = TBLR // NT
EPT = E // NT
CH = 128
CHB = 7
MAXCH = 80
PIECE = 2000


def _flush(table, out_hbm, t, gbase, size):
    @pl.when(t < 12)
    def _():
        pltpu.sync_copy(table.at[pl.ds(t * 208, 208)],
                        out_hbm.at[pl.ds(gbase + t * 208, 208)])
    if size == 2504:
        @pl.when(t == 12)
        def _():
            pltpu.sync_copy(table.at[pl.ds(2496, 8)],
                            out_hbm.at[pl.ds(gbase + 2496, 8)])


def _compact2(t, base, srcv_hbm, dstv_hbm, srcraw, dstraw, bufs):
    (srcA, dstA), (srcB, dstB) = bufs
    baseB = base + QR[0]

    def piece(pi, carry):
        pltpu.sync_copy(srcv_hbm.at[pl.ds(t * EPT + pi * PIECE, PIECE)],
                        srcraw)
        pltpu.sync_copy(dstv_hbm.at[pl.ds(t * EPT + pi * PIECE, PIECE)],
                        dstraw)

        def cbody(j, carry):
            cntA, cntB = carry
            sv = srcraw[pl.ds(j * 16, 16)]
            dv = dstraw[pl.ds(j * 16, 16)]
            dA = dv - base
            mA = (dA >= 0) & (dA < QR[0])
            posA = plsc.cumsum(mA.astype(jnp.int32))
            iA = cntA + posA - 1
            plsc.store_scatter(srcA, [iA >> CHB, iA & (CH - 1)], sv, mask=mA)
            plsc.store_scatter(dstA, [iA >> CHB, iA & (CH - 1)], dA, mask=mA)
            dB = dv - baseB
            mB = (dB >= 0) & (dB < QR[1])
            posB = plsc.cumsum(mB.astype(jnp.int32))
            iB = cntB + posB - 1
            plsc.store_scatter(srcB, [iB >> CHB, iB & (CH - 1)], sv, mask=mB)
            plsc.store_scatter(dstB, [iB >> CHB, iB & (CH - 1)], dB, mask=mB)
            return (cntA + posA[15], cntB + posB[15])

        return lax.fori_loop(0, PIECE // 16, cbody, carry)

    cntA, cntB = lax.fori_loop(0, EPT // PIECE, piece,
                               (jnp.int32(0), jnp.int32(0)))
    lane = lax.iota(jnp.int32, 16)
    for (sbuf, dbuf, cnt, q) in ((srcA, dstA, cntA, 0), (srcB, dstB, cntB, 1)):
        for k in range(CH // 16):
            i2 = cnt + k * 16 + lane
            plsc.store_scatter(sbuf, [i2 >> CHB, i2 & (CH - 1)], lane + k * 16)
            plsc.store_scatter(dbuf, [i2 >> CHB, i2 & (CH - 1)], lane + QR[q])
    return cntA, (cntA + CH - 1) // CH, cntB, (cntB + CH - 1) // CH


SUB = CH // 2


def _scatter_range(t, table, zeros_hbm, tbl_hbm, out_hbm, gbase, size,
                   src2d, dst2d, nch, rows, didx, g0, g1, ssem):
    pltpu.sync_copy(zeros_hbm.at[pl.ds(t * ZPT, ZPT)],
                    table.at[pl.ds(t * ZPT, ZPT)])
    plsc.subcore_barrier()

    def gth(j, half, gsem):
        return pltpu.make_async_copy(
            tbl_hbm.at[src2d.at[j, pl.ds(half * SUB, SUB)]],
            rows.at[pl.ds(half * SUB, SUB)], gsem)

    def scat(j, half):
        for g in range(SUB // 16):
            didx[pl.ds(g * 16, 16)] = dst2d[j, pl.ds(half * SUB + g * 16, 16)]
        pltpu.async_copy(rows.at[pl.ds(half * SUB, SUB)],
                         table.at[didx], ssem, add=True).wait()

    @pl.when(nch > 0)
    def _():
        gth(0, 0, g0).start()

    def mbody(j, carry):
        gth(j, 1, g1).start()
        gth(j, 0, g0).wait()
        scat(j, 0)

        @pl.when(j + 1 < nch)
        def _():
            gth(j + 1, 0, g0).start()

        gth(j, 1, g1).wait()
        scat(j, 1)
        return carry

    lax.fori_loop(0, nch, mbody, 0)
    plsc.subcore_barrier()
    _flush(table, out_hbm, t, gbase, size)
    plsc.subcore_barrier()


def _seg1a_body(xi_hbm, siu_hbm, diu_hbm, zeros_hbm,
                aggu_hbm,
                srcraw, dstraw, srcA, dstA, srcB, dstB, rows, cntv,
                didx, table, g0, g1, ssem):
    c = lax.axis_index("c")
    t = lax.axis_index("s")
    base = c * HALF
    bufs = ((srcA, dstA), (srcB, dstB))

    cA, nchA, cB, nchB = _compact2(t, base, siu_hbm, diu_hbm,
                                   srcraw, dstraw, bufs)
    _scatter_range(t, table, zeros_hbm, xi_hbm, aggu_hbm,
                   base, QR[0], srcA, dstA, nchA, rows, didx, g0, g1, ssem)
    _scatter_range(t, table, zeros_hbm, xi_hbm, aggu_hbm,
                   base + QR[0], QR[1], srcB, dstB, nchB, rows, didx, g0, g1, ssem)


def _seg1b_body(xu_hbm, sui_hbm, dui_hbm, zeros_hbm,
                aggi_hbm, csrc_hbm, cdst_hbm, cnt_hbm,
                srcraw, dstraw, srcA, dstA, srcB, dstB, rows, cntv,
                didx, table, g0, g1, ssem):
    c = lax.axis_index("c")
    t = lax.axis_index("s")
    w = c * NT + t
    base = c * HALF
    bufs = ((srcA, dstA), (srcB, dstB))
    lane = lax.iota(jnp.int32, 16)

    cA, nchA, cB, nchB = _compact2(t, base, sui_hbm, dui_hbm,
                                   srcraw, dstraw, bufs)
    pltpu.sync_copy(srcA, csrc_hbm.at[w * 2])
    pltpu.sync_copy(dstA, cdst_hbm.at[w * 2])
    pltpu.sync_copy(srcB, csrc_hbm.at[w * 2 + 1])
    pltpu.sync_copy(dstB, cdst_hbm.at[w * 2 + 1])
    cntv[...] = jnp.where(lane < 8, cA, cB)
    pltpu.sync_copy(cntv, cnt_hbm.at[w])
    _scatter_range(t, table, zeros_hbm, xu_hbm, aggi_hbm,
                   base, QR[0], srcA, dstA, nchA, rows, didx, g0, g1, ssem)
    _scatter_range(t, table, zeros_hbm, xu_hbm, aggi_hbm,
                   base + QR[0], QR[1], srcB, dstB, nchB, rows, didx, g0, g1, ssem)


def _seg2_body(hu0_hbm, hu1_hbm, csrc_hbm, cdst_hbm, cnt_hbm, zeros_hbm,
               agg20_hbm, agg21_hbm,
               srcA, dstA, srcB, dstB, rows, cntv, didx, table,
               g0, g1, ssem):
    c = lax.axis_index("c")
    t = lax.axis_index("s")
    w = c * NT + t
    base = c * HALF
    pltpu.sync_copy(csrc_hbm.at[w * 2], srcA)
    pltpu.sync_copy(cdst_hbm.at[w * 2], dstA)
    pltpu.sync_copy(csrc_hbm.at[w * 2 + 1], srcB)
    pltpu.sync_copy(cdst_hbm.at[w * 2 + 1], dstB)
    pltpu.sync_copy(cnt_hbm.at[w], cntv)
    cv = cntv[...]
    cA = cv[0]
    cB = cv[8]
    nchA = (cA + CH - 1) // CH
    nchB = (cB + CH - 1) // CH
    for tbl_hbm, out_hbm in ((hu0_hbm, agg20_hbm), (hu1_hbm, agg21_hbm)):
        _scatter_range(t, table, zeros_hbm, tbl_hbm, out_hbm,
                       base, QR[0], srcA, dstA, nchA, rows, didx, g0, g1, ssem)
        _scatter_range(t, table, zeros_hbm, tbl_hbm, out_hbm,
                       base + QR[0], QR[1], srcB, dstB, nchB, rows, didx, g0, g1, ssem)


_SC_MESH = plsc.VectorSubcoreMesh(core_axis_name="c", subcore_axis_name="s")
_SC_PARAMS = pltpu.CompilerParams(needs_layout_passes=False,
                                  use_tc_tiling_on_sc=False)

_SEG_SCRATCH = [pltpu.VMEM((PIECE,), jnp.int32),
                pltpu.VMEM((PIECE,), jnp.int32),
                pltpu.VMEM((MAXCH, CH), jnp.int32),
                pltpu.VMEM((MAXCH, CH), jnp.int32),
                pltpu.VMEM((MAXCH, CH), jnp.int32),
                pltpu.VMEM((MAXCH, CH), jnp.int32),
                pltpu.VMEM((CH, D), jnp.bfloat16),
                pltpu.VMEM((16,), jnp.int32),
                pltpu.VMEM((SUB,), jnp.int32),
                pltpu.VMEM_SHARED((TBLR, D), jnp.bfloat16),
                pltpu.SemaphoreType.DMA,
                pltpu.SemaphoreType.DMA,
                pltpu.SemaphoreType.DMA]

_seg1a = pl.kernel(
    mesh=_SC_MESH,
    compiler_params=_SC_PARAMS,
    out_type=[jax.ShapeDtypeStruct((N, D), jnp.bfloat16)],
    scratch_types=_SEG_SCRATCH,
)(_seg1a_body)

_seg1b = pl.kernel(
    mesh=_SC_MESH,
    compiler_params=_SC_PARAMS,
    out_type=[jax.ShapeDtypeStruct((N, D), jnp.bfloat16),
              jax.ShapeDtypeStruct((NC * NT * 2, MAXCH, CH), jnp.int32),
              jax.ShapeDtypeStruct((NC * NT * 2, MAXCH, CH), jnp.int32),
              jax.ShapeDtypeStruct((NC * NT, 16), jnp.int32)],
    scratch_types=_SEG_SCRATCH,
)(_seg1b_body)

_seg2 = pl.kernel(
    mesh=_SC_MESH,
    compiler_params=_SC_PARAMS,
    out_type=[jax.ShapeDtypeStruct((N, D), jnp.bfloat16),
              jax.ShapeDtypeStruct((N, D), jnp.bfloat16)],
    scratch_types=[pltpu.VMEM((MAXCH, CH), jnp.int32),
                   pltpu.VMEM((MAXCH, CH), jnp.int32),
                   pltpu.VMEM((MAXCH, CH), jnp.int32),
                   pltpu.VMEM((MAXCH, CH), jnp.int32),
                   pltpu.VMEM((CH, D), jnp.bfloat16),
                   pltpu.VMEM((16,), jnp.int32),
                   pltpu.VMEM((SUB,), jnp.int32),
                   pltpu.VMEM_SHARED((TBLR, D), jnp.bfloat16),
                   pltpu.SemaphoreType.DMA,
                   pltpu.SemaphoreType.DMA,
                   pltpu.SemaphoreType.DMA],
)(_seg2_body)


_RB = 2000
_RB2 = 1000


def _xi_body(x_ref, y_ref, emb_ref, o_ref, o16_ref):
    y = y_ref[...]
    w0 = emb_ref[0:1, :]
    w1 = emb_ref[1:2, :]
    add = jnp.where(y == 0, 1.0, 0.0) * w0 + jnp.where(y == 1, 1.0, 0.0) * w1
    xi = x_ref[...] + add
    o_ref[...] = xi
    o16_ref[...] = xi.astype(jnp.bfloat16)


def _xi_call(x_item, y2d, emb_weight):
    return pl.pallas_call(
        _xi_body,
        grid=(N // _RB,),
        in_specs=[pl.BlockSpec((_RB, D), lambda i: (i, 0)),
                  pl.BlockSpec((_RB, 1), lambda i: (i, 0)),
                  pl.BlockSpec((3, D), lambda i: (0, 0))],
        out_specs=[pl.BlockSpec((_RB, D), lambda i: (i, 0)),
                   pl.BlockSpec((_RB, D), lambda i: (i, 0))],
        out_shape=[jax.ShapeDtypeStruct((N, D), jnp.float32),
                   jax.ShapeDtypeStruct((N, D), jnp.bfloat16)],
    )(x_item, y2d, emb_weight)


def _mm_u_body(aggu_ref, xu_ref, wru_ref, wtu_ref, su_ref, bu_ref,
               hu0_ref, hu1_ref):
    pre_u = (jnp.dot(aggu_ref[...].astype(jnp.float32), wru_ref[...],
                     preferred_element_type=jnp.float32)
             + jnp.dot(xu_ref[...], wtu_ref[...],
                       preferred_element_type=jnp.float32))
    hu = jnp.maximum(pre_u * su_ref[...] + bu_ref[...], 0.0)
    hu0_ref[...] = hu[:, :D].astype(jnp.bfloat16)
    hu1_ref[...] = hu[:, D:].astype(jnp.bfloat16)


def _mm_u_call(agg_u, xu, wru, wtu, su, bu):
    blk = lambda r, c: pl.BlockSpec((r, c), lambda i: (i, 0))
    full = lambda r, c: pl.BlockSpec((r, c), lambda i: (0, 0))
    return pl.pallas_call(
        _mm_u_body,
        grid=(N // _RB,),
        in_specs=[blk(_RB, D), blk(_RB, D),
                  full(D, H), full(D, H), full(1, H), full(1, H)],
        out_specs=[blk(_RB, D), blk(_RB, D)],
        out_shape=[jax.ShapeDtypeStruct((N, D), jnp.bfloat16),
                   jax.ShapeDtypeStruct((N, D), jnp.bfloat16)],
    )(agg_u, xu, wru, wtu, su, bu)


def _mm_i_body(aggi_ref, xi_ref, wri_ref, wti_ref, si_ref, bi_ref,
               wa_ref, wb_ref, bias_ref, pp_ref):
    pre_i = (jnp.dot(aggi_ref[...].astype(jnp.float32), wri_ref[...],
                     preferred_element_type=jnp.float32)
             + jnp.dot(xi_ref[...], wti_ref[...],
                       preferred_element_type=jnp.float32))
    hi = jnp.maximum(pre_i * si_ref[...] + bi_ref[...], 0.0)
    acc = jnp.dot(xi_ref[...], wa_ref[...],
                  preferred_element_type=jnp.float32)
    acc += jnp.dot(hi, wb_ref[...], preferred_element_type=jnp.float32)
    pp_ref[...] = acc + bias_ref[...]


def _mm_i_call(agg_i, xi, wri, wti, si, bi, wa, wb, bias):
    blk = lambda r, c: pl.BlockSpec((r, c), lambda i: (i, 0))
    full = lambda r, c: pl.BlockSpec((r, c), lambda i: (0, 0))
    return pl.pallas_call(
        _mm_i_body,
        grid=(N // _RB,),
        in_specs=[blk(_RB, D), blk(_RB, D),
                  full(D, H), full(D, H), full(1, H), full(1, H),
                  full(D, 2), full(H, 2), full(1, 2)],
        out_specs=blk(_RB, 2),
        out_shape=jax.ShapeDtypeStruct((N, 2), jnp.float32),
    )(agg_i, xi, wri, wti, si, bi, wa, wb, bias)


def _fin_body(pp_ref, a0_ref, a1_ref, wc0_ref, wc1_ref, o_ref):
    acc = jnp.dot(a0_ref[...].astype(jnp.float32), wc0_ref[...],
                  preferred_element_type=jnp.float32)
    acc += jnp.dot(a1_ref[...].astype(jnp.float32), wc1_ref[...],
                   preferred_element_type=jnp.float32)
    o_ref[...] = acc + pp_ref[...]


def _fin_call(pp, a0, a1, wc0, wc1):
    blk = lambda r, c: pl.BlockSpec((r, c), lambda i: (i, 0))
    full = lambda r, c: pl.BlockSpec((r, c), lambda i: (0, 0))
    return pl.pallas_call(
        _fin_body,
        grid=(N // _RB,),
        in_specs=[blk(_RB, 2), blk(_RB, D), blk(_RB, D),
                  full(D, 2), full(D, 2)],
        out_specs=blk(_RB, 2),
        out_shape=jax.ShapeDtypeStruct((N, 2), jnp.float32),
    )(pp, a0, a1, wc0, wc1)


def kernel(x_item, x_user, edge_index_iu, edge_index_ui, y_emb, emb_weight,
           W_rel1_iu, b_rel1_iu, W_root1_iu, W_rel1_ui, b_rel1_ui, W_root1_ui,
           gamma_item, beta_item, gamma_user, beta_user,
           W_rel2_iu, b_rel2_iu, W_root2_iu, W_rel2_ui, b_rel2_ui, W_root2_ui,
           lin_W, lin_b):
    inv = 1.0 / jnp.sqrt(1.0 + 1e-5)
    su = (inv * gamma_user).reshape(1, H)
    bu = (b_rel1_iu * inv * gamma_user + beta_user).reshape(1, H)
    si = (inv * gamma_item).reshape(1, H)
    bi = (b_rel1_ui * inv * gamma_item + beta_item).reshape(1, H)
    wc = lin_W[D + H:]
    wa = lin_W[:D]
    wb = lin_W[D:D + H] + W_root2_ui @ wc
    wcf = W_rel2_ui @ wc
    bias = (lin_b + b_rel2_ui @ wc).reshape(1, 2)
    zeros_bf16 = jnp.zeros((TBLR, D), jnp.bfloat16)

    xi, xi16 = _xi_call(x_item, y_emb.reshape(N, 1), emb_weight)
    xu16 = x_user.astype(jnp.bfloat16)
    agg_u, = _seg1a(xi16, edge_index_iu[0], edge_index_iu[1], zeros_bf16)
    agg_i, csrc, cdst, cnts = _seg1b(
        xu16, edge_index_ui[0], edge_index_ui[1], zeros_bf16)
    hu0, hu1 = _mm_u_call(agg_u, x_user, W_rel1_iu, W_root1_iu, su, bu)
    pp = _mm_i_call(agg_i, xi, W_rel1_ui, W_root1_ui, si, bi,
                    wa, wb, bias)
    a0, a1 = _seg2(hu0, hu1, csrc, cdst, cnts, zeros_bf16)
    return _fin_call(pp, a0, a1, wcf[:D], wcf[D:])
